# Initial kernel scaffold; baseline (speedup 1.0000x reference)
import jax
import jax.numpy as jnp
from jax.experimental import pallas as pl


def kernel(features, edge_index, W1, att_src1, att_dst1, W2, gene_att, Wpred, bpred):
    raise NotImplementedError("write your pallas kernel here")


# speedup vs baseline: 1.0503x; 1.0503x over previous
import functools

import jax
import jax.numpy as jnp
from jax import lax
from jax.experimental import pallas as pl
from jax.experimental.pallas import tpu as pltpu

N = 10000
NP = 10240
E = 320000
BLK = 128
GRID = NP // BLK


def _elu(x):
    return jnp.where(x > 0, x, jnp.exp(jnp.minimum(x, 0.0)) - 1.0)


def _tc1_body(f_ref, g_ref, w1_ref, av_ref, x1_ref, asd_ref):
    att = f_ref[...] * g_ref[...]
    x1 = jnp.dot(att, w1_ref[...], preferred_element_type=jnp.float32,
                 precision=lax.Precision.HIGHEST)
    x1_ref[...] = x1
    asd_ref[...] = jnp.dot(x1, av_ref[...], preferred_element_type=jnp.float32,
                           precision=lax.Precision.HIGHEST)


def _tc1(features_p, gene_att, W1, av):
    return pl.pallas_call(
        _tc1_body,
        grid=(GRID,),
        in_specs=[
            pl.BlockSpec((BLK, 128), lambda i: (i, 0)),
            pl.BlockSpec((1, 128), lambda i: (0, 0)),
            pl.BlockSpec((128, 64), lambda i: (0, 0)),
            pl.BlockSpec((64, 2), lambda i: (0, 0)),
        ],
        out_specs=[
            pl.BlockSpec((BLK, 64), lambda i: (i, 0)),
            pl.BlockSpec((BLK, 2), lambda i: (i, 0)),
        ],
        out_shape=[
            jax.ShapeDtypeStruct((NP, 64), jnp.float32),
            jax.ShapeDtypeStruct((NP, 2), jnp.float32),
        ],
    )(features_p, gene_att, W1, av)


def _tc2_body(p0_ref, p1_ref, w2_ref, w2t_ref, wp_ref, bp_ref,
              h2_ref, x3_ref, lp_ref):
    h1 = _elu(p0_ref[...] + p1_ref[...])
    h2 = jnp.dot(h1, w2_ref[...], preferred_element_type=jnp.float32,
                 precision=lax.Precision.HIGHEST)
    h2_ref[...] = h2
    x3_ref[...] = jnp.dot(h2, w2t_ref[...], preferred_element_type=jnp.float32,
                          precision=lax.Precision.HIGHEST)
    pred = jnp.dot(h2, wp_ref[...], preferred_element_type=jnp.float32,
                   precision=lax.Precision.HIGHEST) + bp_ref[...]
    m = jnp.max(pred, axis=-1, keepdims=True)
    lse = jnp.log(jnp.sum(jnp.exp(pred - m), axis=-1, keepdims=True)) + m
    lp_ref[...] = pred - lse


def _tc2(p0, p1, W2, W2t, Wpred, bpred2):
    return pl.pallas_call(
        _tc2_body,
        grid=(GRID,),
        in_specs=[
            pl.BlockSpec((BLK, 64), lambda i: (i, 0)),
            pl.BlockSpec((BLK, 64), lambda i: (i, 0)),
            pl.BlockSpec((64, 32), lambda i: (0, 0)),
            pl.BlockSpec((32, 64), lambda i: (0, 0)),
            pl.BlockSpec((32, 16), lambda i: (0, 0)),
            pl.BlockSpec((1, 16), lambda i: (0, 0)),
        ],
        out_specs=[
            pl.BlockSpec((BLK, 32), lambda i: (i, 0)),
            pl.BlockSpec((BLK, 64), lambda i: (i, 0)),
            pl.BlockSpec((BLK, 16), lambda i: (i, 0)),
        ],
        out_shape=[
            jax.ShapeDtypeStruct((NP, 32), jnp.float32),
            jax.ShapeDtypeStruct((NP, 64), jnp.float32),
            jax.ShapeDtypeStruct((NP, 16), jnp.float32),
        ],
    )(p0, p1, W2, W2t, Wpred, bpred2)


def _tc3_body(p0_ref, p1_ref, w1t_ref, h4_ref):
    h3 = _elu(p0_ref[...] + p1_ref[...])
    h4_ref[...] = jnp.dot(h3, w1t_ref[...], preferred_element_type=jnp.float32,
                          precision=lax.Precision.HIGHEST)


def _tc3(p0, p1, W1t):
    return pl.pallas_call(
        _tc3_body,
        grid=(GRID,),
        in_specs=[
            pl.BlockSpec((BLK, 64), lambda i: (i, 0)),
            pl.BlockSpec((BLK, 64), lambda i: (i, 0)),
            pl.BlockSpec((64, 128), lambda i: (0, 0)),
        ],
        out_specs=pl.BlockSpec((BLK, 128), lambda i: (i, 0)),
        out_shape=jax.ShapeDtypeStruct((NP, 128), jnp.float32),
    )(p0, p1, W1t)


def _leaky_relu(x, slope=0.2):
    return jnp.where(x >= 0, x, slope * x)


def _edges_jnp(a_s, a_d, src, dst, x1, x3):
    e = _leaky_relu(a_s[src] + a_d[dst])
    emax = jax.ops.segment_max(e, dst, num_segments=N)
    emax = jnp.where(jnp.isfinite(emax), emax, 0.0)
    ex = jnp.exp(e - emax[dst])
    esum = jax.ops.segment_sum(ex, dst, num_segments=N)
    alpha = ex / (esum[dst] + 1e-16)
    h1 = jax.ops.segment_sum(x1[src] * alpha[:, None], dst, num_segments=N)
    h3f = lambda x3_: jax.ops.segment_sum(x3_[src] * alpha[:, None], dst, num_segments=N)
    return h1, alpha, h3f


def kernel(features, edge_index, W1, att_src1, att_dst1, W2, gene_att, Wpred, bpred):
    src = edge_index[0]
    dst = edge_index[1]
    features_p = jnp.pad(features, ((0, NP - N), (0, 0)))
    av = jnp.stack([att_src1, att_dst1], axis=1)
    x1p, asd = _tc1(features_p, gene_att, W1, av)
    a_s = asd[:N, 0]
    a_d = asd[:N, 1]
    x1 = x1p[:N]

    h1_full, alpha, h3f = _edges_jnp(a_s, a_d, src, dst, x1, None)
    h1p0 = jnp.pad(h1_full, ((0, NP - N), (0, 0)))
    h1p1 = jnp.zeros_like(h1p0)

    h2p, x3p, lpp = _tc2(h1p0, h1p1, W2, W2.T, Wpred, bpred[None, :])
    h3_full = h3f(x3p[:N])
    h3p0 = jnp.pad(h3_full, ((0, NP - N), (0, 0)))
    h3p1 = jnp.zeros_like(h3p0)
    h4p = _tc3(h3p0, h3p1, W1.T)

    return (h2p[:N], h4p[:N], lpp[:N])


# speedup vs baseline: 11.1149x; 10.5821x over previous
import functools

import jax
import jax.numpy as jnp
from jax import lax
from jax.experimental import pallas as pl
from jax.experimental.pallas import tpu as pltpu
from jax.experimental.pallas import tpu_sc as plsc

N = 10000
NP = 10240
E = 320000
BLK = 128
GRID = NP // BLK

NC = 2
NS = 16
NW = NC * NS
CHUNK = 128
RROWS = E // CHUNK
TROWS = 80
EP = NW * TROWS * CHUNK
NSLC = NP // NS


def _elu(x):
    return jnp.where(x > 0, x, jnp.exp(jnp.minimum(x, 0.0)) - 1.0)


def _tc1_body(f_ref, g_ref, w1_ref, av_ref, x1_ref, asd_ref):
    att = f_ref[...] * g_ref[...]
    x1 = jnp.dot(att, w1_ref[...], preferred_element_type=jnp.float32,
                 precision=lax.Precision.HIGHEST)
    x1_ref[...] = x1
    asd_ref[...] = jnp.dot(x1, av_ref[...], preferred_element_type=jnp.float32,
                           precision=lax.Precision.HIGHEST)


def _tc1(features_p, gene_att, W1, av):
    return pl.pallas_call(
        _tc1_body,
        grid=(GRID,),
        in_specs=[
            pl.BlockSpec((BLK, 128), lambda i: (i, 0)),
            pl.BlockSpec((1, 128), lambda i: (0, 0)),
            pl.BlockSpec((128, 64), lambda i: (0, 0)),
            pl.BlockSpec((64, 2), lambda i: (0, 0)),
        ],
        out_specs=[
            pl.BlockSpec((BLK, 64), lambda i: (i, 0)),
            pl.BlockSpec((BLK, 2), lambda i: (i, 0)),
        ],
        out_shape=[
            jax.ShapeDtypeStruct((NP, 64), jnp.float32),
            jax.ShapeDtypeStruct((NP, 2), jnp.float32),
        ],
    )(features_p, gene_att, W1, av)


def _tc2_body(p0_ref, p1_ref, w2_ref, w2t_ref, wp_ref, bp_ref,
              h2_ref, x3_ref, lp_ref):
    h1 = _elu(p0_ref[...] + p1_ref[...])
    h2 = jnp.dot(h1, w2_ref[...], preferred_element_type=jnp.float32,
                 precision=lax.Precision.HIGHEST)
    h2_ref[...] = h2
    x3_ref[...] = jnp.dot(h2, w2t_ref[...], preferred_element_type=jnp.float32,
                          precision=lax.Precision.HIGHEST)
    pred = jnp.dot(h2, wp_ref[...], preferred_element_type=jnp.float32,
                   precision=lax.Precision.HIGHEST) + bp_ref[...]
    m = jnp.max(pred, axis=-1, keepdims=True)
    lse = jnp.log(jnp.sum(jnp.exp(pred - m), axis=-1, keepdims=True)) + m
    lp_ref[...] = pred - lse


def _tc2(p0, p1, W2, W2t, Wpred, bpred2):
    return pl.pallas_call(
        _tc2_body,
        grid=(GRID,),
        in_specs=[
            pl.BlockSpec((BLK, 64), lambda i: (i, 0)),
            pl.BlockSpec((BLK, 64), lambda i: (i, 0)),
            pl.BlockSpec((64, 32), lambda i: (0, 0)),
            pl.BlockSpec((32, 64), lambda i: (0, 0)),
            pl.BlockSpec((32, 16), lambda i: (0, 0)),
            pl.BlockSpec((1, 16), lambda i: (0, 0)),
        ],
        out_specs=[
            pl.BlockSpec((BLK, 32), lambda i: (i, 0)),
            pl.BlockSpec((BLK, 64), lambda i: (i, 0)),
            pl.BlockSpec((BLK, 16), lambda i: (i, 0)),
        ],
        out_shape=[
            jax.ShapeDtypeStruct((NP, 32), jnp.float32),
            jax.ShapeDtypeStruct((NP, 64), jnp.float32),
            jax.ShapeDtypeStruct((NP, 16), jnp.float32),
        ],
    )(p0, p1, W2, W2t, Wpred, bpred2)


def _tc3_body(p0_ref, p1_ref, w1t_ref, h4_ref):
    h3 = _elu(p0_ref[...] + p1_ref[...])
    h4_ref[...] = jnp.dot(h3, w1t_ref[...], preferred_element_type=jnp.float32,
                          precision=lax.Precision.HIGHEST)


def _tc3(p0, p1, W1t):
    return pl.pallas_call(
        _tc3_body,
        grid=(GRID,),
        in_specs=[
            pl.BlockSpec((BLK, 64), lambda i: (i, 0)),
            pl.BlockSpec((BLK, 64), lambda i: (i, 0)),
            pl.BlockSpec((64, 128), lambda i: (0, 0)),
        ],
        out_specs=pl.BlockSpec((BLK, 128), lambda i: (i, 0)),
        out_shape=jax.ShapeDtypeStruct((NP, 128), jnp.float32),
    )(p0, p1, W1t)


_SC_MESH = plsc.VectorSubcoreMesh(core_axis_name="c", subcore_axis_name="s",
                                  num_cores=NC, num_subcores=NS)
_SC_PARAMS = pltpu.CompilerParams(use_tc_tiling_on_sc=False,
                                  needs_layout_passes=False)


def _wid():
    return lax.axis_index("s") * NC + lax.axis_index("c")


def _tile_max(ref, scratch16):
    def body(i, m):
        return jnp.maximum(m, ref[pl.ds(i * 16, 16)])
    m16 = lax.fori_loop(0, NP // 16, body, jnp.full((16,), -jnp.inf, jnp.float32))
    lanes = lax.iota(jnp.int32, 16)
    for step in (8, 4, 2, 1):
        scratch16[pl.ds(0, 16)] = m16
        idx = jnp.bitwise_and(lanes + step, 15)
        m16 = jnp.maximum(m16, plsc.load_gather(scratch16, [idx]))
    return m16


def _sc_logits_body(src_hbm, dst_hbm, as_hbm, ad_hbm, ex_hbm, esum_hbm,
                    as_v, ad_v, src_v, dst_v, ex_v, zb_v, esum_sh):
    c = lax.axis_index("c")
    s = lax.axis_index("s")
    wid = _wid()
    pltpu.sync_copy(as_hbm, as_v)
    pltpu.sync_copy(ad_hbm, ad_v)
    def zb(i, _):
        zb_v[pl.ds(i * 16, 16)] = jnp.zeros((16,), jnp.float32)
        return 0
    lax.fori_loop(0, NSLC // 16, zb, 0)
    pltpu.sync_copy(zb_v, esum_sh.at[pl.ds(s * NSLC, NSLC)])
    shift_raw = _tile_max(as_v, ex_v) + _tile_max(ad_v, ex_v)
    shift = jnp.where(shift_raw >= 0, shift_raw, 0.2 * shift_raw)
    plsc.subcore_barrier()

    def chunk(j, _):
        row = wid * TROWS + j

        @pl.when(row < RROWS)
        def _():
            base = row * CHUNK
            pltpu.sync_copy(src_hbm.at[pl.ds(base, CHUNK)], src_v)
            pltpu.sync_copy(dst_hbm.at[pl.ds(base, CHUNK)], dst_v)
            for k in range(CHUNK // 16):
                si = src_v[pl.ds(k * 16, 16)]
                di = dst_v[pl.ds(k * 16, 16)]
                e = plsc.load_gather(as_v, [si]) + plsc.load_gather(ad_v, [di])
                e = jnp.where(e >= 0, e, 0.2 * e) - shift
                ex_v[pl.ds(k * 16, 16)] = jnp.exp(e)
            pltpu.sync_copy(ex_v, ex_hbm.at[pl.ds(base, CHUNK)])
            pltpu.sync_copy(ex_v, esum_sh.at[dst_v], add=True)
        return 0

    lax.fori_loop(0, TROWS, chunk, 0)
    plsc.subcore_barrier()
    pltpu.sync_copy(esum_sh.at[pl.ds(s * NSLC, NSLC)],
                    esum_hbm.at[pl.ds(c * NP + s * NSLC, NSLC)])


_sc_logits = pl.kernel(
    _sc_logits_body,
    out_type=[
        jax.ShapeDtypeStruct((EP,), jnp.float32),
        jax.ShapeDtypeStruct((NC * NP,), jnp.float32),
    ],
    mesh=_SC_MESH,
    compiler_params=_SC_PARAMS,
    scratch_types=[
        pltpu.VMEM((NP,), jnp.float32),
        pltpu.VMEM((NP,), jnp.float32),
        pltpu.VMEM((CHUNK,), jnp.int32),
        pltpu.VMEM((CHUNK,), jnp.int32),
        pltpu.VMEM((CHUNK,), jnp.float32),
        pltpu.VMEM((NSLC,), jnp.float32),
        pltpu.VMEM_SHARED((NP,), jnp.float32),
    ],
)


def _zero_rows(buf_v, nrows):
    def zr(i, _):
        for k in range(4):
            buf_v[i, pl.ds(k * 16, 16)] = jnp.zeros((16,), jnp.float32)
        return 0
    lax.fori_loop(0, nrows, zr, 0)


def _scale_rows(rows_v, al_v):
    def sr(e, _):
        idx = jnp.full((16,), e, jnp.int32)
        ab = plsc.load_gather(al_v, [idx])
        for k in range(4):
            rows_v[e, pl.ds(k * 16, 16)] = rows_v[e, pl.ds(k * 16, 16)] * ab
        return 0
    lax.fori_loop(0, CHUNK, sr, 0)


def _sc_aggr1_body(x_hbm, src_hbm, dst_hbm, ex_hbm, esp_hbm,
                   h_hbm, al_hbm,
                   esum_v, tmp_v, src_v, dst_v, ex_v, al_v, rows_v, h_sh, sem):
    c = lax.axis_index("c")
    s = lax.axis_index("s")
    wid = _wid()
    pltpu.sync_copy(esp_hbm.at[pl.ds(0, NP)], esum_v)
    pltpu.sync_copy(esp_hbm.at[pl.ds(NP, NP)], tmp_v)
    def addv(i, _):
        esum_v[pl.ds(i * 16, 16)] = (esum_v[pl.ds(i * 16, 16)]
                                     + tmp_v[pl.ds(i * 16, 16)])
        return 0
    lax.fori_loop(0, NP // 16, addv, 0)
    _zero_rows(rows_v, CHUNK)
    for r in range(NSLC // CHUNK):
        pltpu.sync_copy(rows_v, h_sh.at[pl.ds(s * NSLC + r * CHUNK, CHUNK)])
    plsc.subcore_barrier()

    def chunk(j, _):
        row = wid * TROWS + j

        @pl.when(row < RROWS)
        def _():
            base = row * CHUNK
            pltpu.sync_copy(src_hbm.at[pl.ds(base, CHUNK)], src_v)
            pltpu.sync_copy(dst_hbm.at[pl.ds(base, CHUNK)], dst_v)
            pltpu.sync_copy(ex_hbm.at[pl.ds(base, CHUNK)], ex_v)
            cp = pltpu.async_copy(x_hbm.at[src_v], rows_v, sem)
            for k in range(CHUNK // 16):
                di = dst_v[pl.ds(k * 16, 16)]
                es = plsc.load_gather(esum_v, [di])
                al_v[pl.ds(k * 16, 16)] = ex_v[pl.ds(k * 16, 16)] / (es + 1e-16)
            pltpu.sync_copy(al_v, al_hbm.at[pl.ds(base, CHUNK)])
            cp.wait()
            _scale_rows(rows_v, al_v)
            pltpu.sync_copy(rows_v, h_sh.at[dst_v], add=True)
        return 0

    lax.fori_loop(0, TROWS, chunk, 0)
    plsc.subcore_barrier()
    pltpu.sync_copy(h_sh.at[pl.ds(s * NSLC, NSLC)],
                    h_hbm.at[pl.ds(c * NP + s * NSLC, NSLC)])


_sc_aggr1 = pl.kernel(
    _sc_aggr1_body,
    out_type=[
        jax.ShapeDtypeStruct((NC * NP, 64), jnp.float32),
        jax.ShapeDtypeStruct((EP,), jnp.float32),
    ],
    mesh=_SC_MESH,
    compiler_params=_SC_PARAMS,
    scratch_types=[
        pltpu.VMEM((NP,), jnp.float32),
        pltpu.VMEM((NP,), jnp.float32),
        pltpu.VMEM((CHUNK,), jnp.int32),
        pltpu.VMEM((CHUNK,), jnp.int32),
        pltpu.VMEM((CHUNK,), jnp.float32),
        pltpu.VMEM((CHUNK,), jnp.float32),
        pltpu.VMEM((CHUNK, 64), jnp.float32),
        pltpu.VMEM_SHARED((NP, 64), jnp.float32),
        pltpu.SemaphoreType.DMA,
    ],
)


def _sc_aggr2_body(x_hbm, src_hbm, dst_hbm, al_hbm,
                   h_hbm,
                   src_v, dst_v, al_v, rows_v, h_sh, sem):
    c = lax.axis_index("c")
    s = lax.axis_index("s")
    wid = _wid()
    _zero_rows(rows_v, CHUNK)
    for r in range(NSLC // CHUNK):
        pltpu.sync_copy(rows_v, h_sh.at[pl.ds(s * NSLC + r * CHUNK, CHUNK)])
    plsc.subcore_barrier()

    def chunk(j, _):
        row = wid * TROWS + j

        @pl.when(row < RROWS)
        def _():
            base = row * CHUNK
            pltpu.sync_copy(src_hbm.at[pl.ds(base, CHUNK)], src_v)
            pltpu.sync_copy(dst_hbm.at[pl.ds(base, CHUNK)], dst_v)
            pltpu.sync_copy(al_hbm.at[pl.ds(base, CHUNK)], al_v)
            cp = pltpu.async_copy(x_hbm.at[src_v], rows_v, sem)
            cp.wait()
            _scale_rows(rows_v, al_v)
            pltpu.sync_copy(rows_v, h_sh.at[dst_v], add=True)
        return 0

    lax.fori_loop(0, TROWS, chunk, 0)
    plsc.subcore_barrier()
    pltpu.sync_copy(h_sh.at[pl.ds(s * NSLC, NSLC)],
                    h_hbm.at[pl.ds(c * NP + s * NSLC, NSLC)])


_sc_aggr2 = pl.kernel(
    _sc_aggr2_body,
    out_type=jax.ShapeDtypeStruct((NC * NP, 64), jnp.float32),
    mesh=_SC_MESH,
    compiler_params=_SC_PARAMS,
    scratch_types=[
        pltpu.VMEM((CHUNK,), jnp.int32),
        pltpu.VMEM((CHUNK,), jnp.int32),
        pltpu.VMEM((CHUNK,), jnp.float32),
        pltpu.VMEM((CHUNK, 64), jnp.float32),
        pltpu.VMEM_SHARED((NP, 64), jnp.float32),
        pltpu.SemaphoreType.DMA,
    ],
)


def kernel(features, edge_index, W1, att_src1, att_dst1, W2, gene_att, Wpred, bpred):
    src = edge_index[0].astype(jnp.int32)
    dst = edge_index[1].astype(jnp.int32)
    srcF = jnp.pad(src, (0, EP - E))
    dstF = jnp.pad(dst, (0, EP - E))
    features_p = jnp.pad(features, ((0, NP - N), (0, 0)))
    av = jnp.stack([att_src1, att_dst1], axis=1)

    x1p, asd = _tc1(features_p, gene_att, W1, av)
    a_s = asd[:, 0]
    a_d = asd[:, 1]

    exF, esumP = _sc_logits(srcF, dstF, a_s, a_d)
    h1parts, alphaF = _sc_aggr1(x1p, srcF, dstF, exF, esumP)
    h2p, x3p, lpp = _tc2(h1parts[:NP], h1parts[NP:], W2, W2.T, Wpred, bpred[None, :])
    h3parts = _sc_aggr2(x3p, srcF, dstF, alphaF)
    h4p = _tc3(h3parts[:NP], h3parts[NP:], W1.T)

    return (h2p[:N], h4p[:N], lpp[:N])


# speedup vs baseline: 14.6719x; 1.3200x over previous
import functools

import jax
import jax.numpy as jnp
from jax import lax
from jax.experimental import pallas as pl
from jax.experimental.pallas import tpu as pltpu
from jax.experimental.pallas import tpu_sc as plsc

N = 10000
NP = 10240
E = 320000
BLK = 128
GRID = NP // BLK

NC = 2
NS = 16
NW = NC * NS
CHUNK = 128
RROWS = E // CHUNK
TROWS = 80
EP = NW * TROWS * CHUNK
NSLC = NP // NS


def _elu(x):
    return jnp.where(x > 0, x, jnp.exp(jnp.minimum(x, 0.0)) - 1.0)


def _tc1_body(f_ref, g_ref, w1_ref, av_ref, x1_ref, asd_ref):
    att = f_ref[...] * g_ref[...]
    x1 = jnp.dot(att, w1_ref[...], preferred_element_type=jnp.float32,
                 precision=lax.Precision.HIGHEST)
    x1_ref[...] = x1
    asd_ref[...] = jnp.dot(x1, av_ref[...], preferred_element_type=jnp.float32,
                           precision=lax.Precision.HIGHEST)


def _tc1(features_p, gene_att, W1, av):
    return pl.pallas_call(
        _tc1_body,
        grid=(GRID,),
        in_specs=[
            pl.BlockSpec((BLK, 128), lambda i: (i, 0)),
            pl.BlockSpec((1, 128), lambda i: (0, 0)),
            pl.BlockSpec((128, 64), lambda i: (0, 0)),
            pl.BlockSpec((64, 2), lambda i: (0, 0)),
        ],
        out_specs=[
            pl.BlockSpec((BLK, 64), lambda i: (i, 0)),
            pl.BlockSpec((BLK, 2), lambda i: (i, 0)),
        ],
        out_shape=[
            jax.ShapeDtypeStruct((NP, 64), jnp.float32),
            jax.ShapeDtypeStruct((NP, 2), jnp.float32),
        ],
    )(features_p, gene_att, W1, av)


def _tc2_body(p0_ref, p1_ref, e0_ref, e1_ref, w2_ref, w2t_ref, wp_ref, bp_ref,
              h2_ref, x3_ref, lp_ref):
    den = e0_ref[...] + e1_ref[...] + 1e-16
    h1 = _elu((p0_ref[...] + p1_ref[...]) / den)
    h2 = jnp.dot(h1, w2_ref[...], preferred_element_type=jnp.float32,
                 precision=lax.Precision.HIGHEST)
    h2_ref[...] = h2
    x3_ref[...] = jnp.dot(h2, w2t_ref[...], preferred_element_type=jnp.float32,
                          precision=lax.Precision.HIGHEST)
    pred = jnp.dot(h2, wp_ref[...], preferred_element_type=jnp.float32,
                   precision=lax.Precision.HIGHEST) + bp_ref[...]
    m = jnp.max(pred, axis=-1, keepdims=True)
    lse = jnp.log(jnp.sum(jnp.exp(pred - m), axis=-1, keepdims=True)) + m
    lp_ref[...] = pred - lse


def _tc2(p0, p1, e0, e1, W2, W2t, Wpred, bpred2):
    return pl.pallas_call(
        _tc2_body,
        grid=(GRID,),
        in_specs=[
            pl.BlockSpec((BLK, 64), lambda i: (i, 0)),
            pl.BlockSpec((BLK, 64), lambda i: (i, 0)),
            pl.BlockSpec((BLK, 1), lambda i: (i, 0)),
            pl.BlockSpec((BLK, 1), lambda i: (i, 0)),
            pl.BlockSpec((64, 32), lambda i: (0, 0)),
            pl.BlockSpec((32, 64), lambda i: (0, 0)),
            pl.BlockSpec((32, 16), lambda i: (0, 0)),
            pl.BlockSpec((1, 16), lambda i: (0, 0)),
        ],
        out_specs=[
            pl.BlockSpec((BLK, 32), lambda i: (i, 0)),
            pl.BlockSpec((BLK, 64), lambda i: (i, 0)),
            pl.BlockSpec((BLK, 16), lambda i: (i, 0)),
        ],
        out_shape=[
            jax.ShapeDtypeStruct((NP, 32), jnp.float32),
            jax.ShapeDtypeStruct((NP, 64), jnp.float32),
            jax.ShapeDtypeStruct((NP, 16), jnp.float32),
        ],
    )(p0, p1, e0, e1, W2, W2t, Wpred, bpred2)


def _tc3_body(p0_ref, p1_ref, e0_ref, e1_ref, w1t_ref, h4_ref):
    den = e0_ref[...] + e1_ref[...] + 1e-16
    h3 = _elu((p0_ref[...] + p1_ref[...]) / den)
    h4_ref[...] = jnp.dot(h3, w1t_ref[...], preferred_element_type=jnp.float32,
                          precision=lax.Precision.HIGHEST)


def _tc3(p0, p1, e0, e1, W1t):
    return pl.pallas_call(
        _tc3_body,
        grid=(GRID,),
        in_specs=[
            pl.BlockSpec((BLK, 64), lambda i: (i, 0)),
            pl.BlockSpec((BLK, 64), lambda i: (i, 0)),
            pl.BlockSpec((BLK, 1), lambda i: (i, 0)),
            pl.BlockSpec((BLK, 1), lambda i: (i, 0)),
            pl.BlockSpec((64, 128), lambda i: (0, 0)),
        ],
        out_specs=pl.BlockSpec((BLK, 128), lambda i: (i, 0)),
        out_shape=jax.ShapeDtypeStruct((NP, 128), jnp.float32),
    )(p0, p1, e0, e1, W1t)


_SC_MESH = plsc.VectorSubcoreMesh(core_axis_name="c", subcore_axis_name="s",
                                  num_cores=NC, num_subcores=NS)
_SC_PARAMS = pltpu.CompilerParams(use_tc_tiling_on_sc=False,
                                  needs_layout_passes=False)


def _wid():
    return lax.axis_index("s") * NC + lax.axis_index("c")


def _tile_max(ref, scratch16):
    def body(i, m):
        return jnp.maximum(m, ref[pl.ds(i * 16, 16)])
    m16 = lax.fori_loop(0, NP // 16, body, jnp.full((16,), -jnp.inf, jnp.float32))
    lanes = lax.iota(jnp.int32, 16)
    for step in (8, 4, 2, 1):
        scratch16[pl.ds(0, 16)] = m16
        idx = jnp.bitwise_and(lanes + step, 15)
        m16 = jnp.maximum(m16, plsc.load_gather(scratch16, [idx]))
    return m16


def _sc_pass_body(emit_aux, *refs):
    if emit_aux:
        (x_hbm, ei_hbm, as_hbm, ad_hbm, h_hbm, esum_hbm,
         as_v, ad_v, sd_v, ex_v, rows_v, h_sh, esum_sh, sem) = refs
    else:
        (x_hbm, ei_hbm, as_hbm, ad_hbm, h_hbm,
         as_v, ad_v, sd_v, ex_v, rows_v, h_sh, sem) = refs
    c = lax.axis_index("c")
    s = lax.axis_index("s")
    wid = _wid()
    pltpu.sync_copy(as_hbm, as_v)
    pltpu.sync_copy(ad_hbm, ad_v)
    def zr(i, _):
        for k in range(4):
            rows_v[i, pl.ds(k * 16, 16)] = jnp.zeros((16,), jnp.float32)
        return 0
    lax.fori_loop(0, CHUNK, zr, 0)
    for r in range(NSLC // CHUNK):
        pltpu.sync_copy(rows_v, h_sh.at[pl.ds(s * NSLC + r * CHUNK, CHUNK)])
    if emit_aux:
        def ze(i, _):
            ex_v[pl.ds(i * 16, 16)] = jnp.zeros((16,), jnp.float32)
            return 0
        lax.fori_loop(0, CHUNK // 16, ze, 0)
        for r in range(NSLC // CHUNK):
            pltpu.sync_copy(ex_v, esum_sh.at[pl.ds(s * NSLC + r * CHUNK, CHUNK)])
    shift_raw = _tile_max(as_v, ex_v) + _tile_max(ad_v, ex_v)
    shift = jnp.where(shift_raw >= 0, shift_raw, 0.2 * shift_raw)
    plsc.subcore_barrier()

    def chunk(j, _):
        row = wid * TROWS + j

        @pl.when(row < RROWS)
        def _():
            pltpu.sync_copy(ei_hbm.at[row], sd_v)
            cp = pltpu.async_copy(x_hbm.at[sd_v.at[0]], rows_v, sem)
            for k in range(CHUNK // 16):
                si = sd_v[0, pl.ds(k * 16, 16)]
                di = sd_v[1, pl.ds(k * 16, 16)]
                e = plsc.load_gather(as_v, [si]) + plsc.load_gather(ad_v, [di])
                e = jnp.where(e >= 0, e, 0.2 * e) - shift
                ex_v[pl.ds(k * 16, 16)] = jnp.exp(e)
            if emit_aux:
                pltpu.sync_copy(ex_v, esum_sh.at[sd_v.at[1]], add=True)
            cp.wait()
            def sc4(i, _):
                for u in range(4):
                    e4 = i * 4 + u
                    ab = plsc.load_gather(ex_v, [jnp.full((16,), e4, jnp.int32)])
                    for k in range(4):
                        rows_v[e4, pl.ds(k * 16, 16)] = (
                            rows_v[e4, pl.ds(k * 16, 16)] * ab)
                return 0
            lax.fori_loop(0, CHUNK // 4, sc4, 0)
            pltpu.sync_copy(rows_v, h_sh.at[sd_v.at[1]], add=True)
        return 0

    lax.fori_loop(0, TROWS, chunk, 0)
    plsc.subcore_barrier()
    pltpu.sync_copy(h_sh.at[pl.ds(s * NSLC, NSLC)],
                    h_hbm.at[pl.ds(c * NP + s * NSLC, NSLC)])
    if emit_aux:
        pltpu.sync_copy(esum_sh.at[pl.ds(s * NSLC, NSLC)],
                        esum_hbm.at[pl.ds(c * NP + s * NSLC, NSLC)])


_COMMON_SCRATCH = [
    pltpu.VMEM((NP,), jnp.float32),
    pltpu.VMEM((NP,), jnp.float32),
    pltpu.VMEM((2, CHUNK), jnp.int32),
    pltpu.VMEM((CHUNK,), jnp.float32),
    pltpu.VMEM((CHUNK, 64), jnp.float32),
    pltpu.VMEM_SHARED((NP, 64), jnp.float32),
]

_sc_pass1 = pl.kernel(
    functools.partial(_sc_pass_body, True),
    out_type=[
        jax.ShapeDtypeStruct((NC * NP, 64), jnp.float32),
        jax.ShapeDtypeStruct((NC * NP,), jnp.float32),
    ],
    mesh=_SC_MESH,
    compiler_params=_SC_PARAMS,
    scratch_types=_COMMON_SCRATCH + [
        pltpu.VMEM_SHARED((NP,), jnp.float32),
        pltpu.SemaphoreType.DMA,
    ],
)

_sc_pass2 = pl.kernel(
    functools.partial(_sc_pass_body, False),
    out_type=jax.ShapeDtypeStruct((NC * NP, 64), jnp.float32),
    mesh=_SC_MESH,
    compiler_params=_SC_PARAMS,
    scratch_types=_COMMON_SCRATCH + [
        pltpu.SemaphoreType.DMA,
    ],
)


def kernel(features, edge_index, W1, att_src1, att_dst1, W2, gene_att, Wpred, bpred):
    src = edge_index[0].astype(jnp.int32)
    dst = edge_index[1].astype(jnp.int32)
    nrt = EP // CHUNK
    ei3 = jnp.concatenate(
        [jnp.pad(src, (0, EP - E)).reshape(nrt, 1, CHUNK),
         jnp.pad(dst, (0, EP - E)).reshape(nrt, 1, CHUNK)], axis=1)
    features_p = jnp.pad(features, ((0, NP - N), (0, 0)))
    av = jnp.stack([att_src1, att_dst1], axis=1)

    x1p, asd = _tc1(features_p, gene_att, W1, av)
    a_s = asd[:, 0]
    a_d = asd[:, 1]

    h1parts, esumP = _sc_pass1(x1p, ei3, a_s, a_d)
    e0 = esumP[:NP, None]
    e1 = esumP[NP:, None]
    h2p, x3p, lpp = _tc2(h1parts[:NP], h1parts[NP:], e0, e1,
                         W2, W2.T, Wpred, bpred[None, :])
    h3parts = _sc_pass2(x3p, ei3, a_s, a_d)
    h4p = _tc3(h3parts[:NP], h3parts[NP:], e0, e1, W1.T)

    return (h2p[:N], h4p[:N], lpp[:N])


# speedup vs baseline: 20.5920x; 1.4035x over previous
import functools

import jax
import jax.numpy as jnp
from jax import lax
from jax.experimental import pallas as pl
from jax.experimental.pallas import tpu as pltpu
from jax.experimental.pallas import tpu_sc as plsc

N = 10000
NP = 10240
E = 320000
BLK = 128
GRID = NP // BLK

NC = 2
NS = 16
NW = NC * NS
CHUNK = 128
RROWS = E // CHUNK
TROWS = 80
EP = NW * TROWS * CHUNK
NSLC = NP // NS


def _elu(x):
    return jnp.where(x > 0, x, jnp.exp(jnp.minimum(x, 0.0)) - 1.0)


def _tc1_body(f_ref, g_ref, w1_ref, av_ref, x1_ref, asd_ref):
    att = f_ref[...] * g_ref[...]
    x1 = jnp.dot(att, w1_ref[...], preferred_element_type=jnp.float32,
                 precision=lax.Precision.HIGHEST)
    x1_ref[...] = x1
    asd_ref[...] = jnp.dot(x1, av_ref[...], preferred_element_type=jnp.float32,
                           precision=lax.Precision.HIGHEST)


def _tc1(features_p, gene_att, W1, av):
    return pl.pallas_call(
        _tc1_body,
        grid=(GRID,),
        in_specs=[
            pl.BlockSpec((BLK, 128), lambda i: (i, 0)),
            pl.BlockSpec((1, 128), lambda i: (0, 0)),
            pl.BlockSpec((128, 64), lambda i: (0, 0)),
            pl.BlockSpec((64, 2), lambda i: (0, 0)),
        ],
        out_specs=[
            pl.BlockSpec((BLK, 64), lambda i: (i, 0)),
            pl.BlockSpec((BLK, 2), lambda i: (i, 0)),
        ],
        out_shape=[
            jax.ShapeDtypeStruct((NP, 64), jnp.float32),
            jax.ShapeDtypeStruct((NP, 2), jnp.float32),
        ],
    )(features_p, gene_att, W1, av)


def _tc2_body(p0_ref, p1_ref, e0_ref, e1_ref, w2_ref, w2t_ref, wp_ref, bp_ref,
              h2_ref, x3_ref, lp_ref):
    den = e0_ref[...] + e1_ref[...] + 1e-16
    h1 = _elu((p0_ref[...] + p1_ref[...]) / den)
    h2 = jnp.dot(h1, w2_ref[...], preferred_element_type=jnp.float32,
                 precision=lax.Precision.HIGHEST)
    h2_ref[...] = h2
    x3_ref[...] = jnp.dot(h2, w2t_ref[...], preferred_element_type=jnp.float32,
                          precision=lax.Precision.HIGHEST)
    pred = jnp.dot(h2, wp_ref[...], preferred_element_type=jnp.float32,
                   precision=lax.Precision.HIGHEST) + bp_ref[...]
    m = jnp.max(pred, axis=-1, keepdims=True)
    lse = jnp.log(jnp.sum(jnp.exp(pred - m), axis=-1, keepdims=True)) + m
    lp_ref[...] = pred - lse


def _tc2(p0, p1, e0, e1, W2, W2t, Wpred, bpred2):
    return pl.pallas_call(
        _tc2_body,
        grid=(GRID,),
        in_specs=[
            pl.BlockSpec((BLK, 64), lambda i: (i, 0)),
            pl.BlockSpec((BLK, 64), lambda i: (i, 0)),
            pl.BlockSpec((BLK, 1), lambda i: (i, 0)),
            pl.BlockSpec((BLK, 1), lambda i: (i, 0)),
            pl.BlockSpec((64, 32), lambda i: (0, 0)),
            pl.BlockSpec((32, 64), lambda i: (0, 0)),
            pl.BlockSpec((32, 16), lambda i: (0, 0)),
            pl.BlockSpec((1, 16), lambda i: (0, 0)),
        ],
        out_specs=[
            pl.BlockSpec((BLK, 32), lambda i: (i, 0)),
            pl.BlockSpec((BLK, 64), lambda i: (i, 0)),
            pl.BlockSpec((BLK, 16), lambda i: (i, 0)),
        ],
        out_shape=[
            jax.ShapeDtypeStruct((NP, 32), jnp.float32),
            jax.ShapeDtypeStruct((NP, 64), jnp.float32),
            jax.ShapeDtypeStruct((NP, 16), jnp.float32),
        ],
    )(p0, p1, e0, e1, W2, W2t, Wpred, bpred2)


def _tc3_body(p0_ref, p1_ref, e0_ref, e1_ref, w1t_ref, h4_ref):
    den = e0_ref[...] + e1_ref[...] + 1e-16
    h3 = _elu((p0_ref[...] + p1_ref[...]) / den)
    h4_ref[...] = jnp.dot(h3, w1t_ref[...], preferred_element_type=jnp.float32,
                          precision=lax.Precision.HIGHEST)


def _tc3(p0, p1, e0, e1, W1t):
    return pl.pallas_call(
        _tc3_body,
        grid=(GRID,),
        in_specs=[
            pl.BlockSpec((BLK, 64), lambda i: (i, 0)),
            pl.BlockSpec((BLK, 64), lambda i: (i, 0)),
            pl.BlockSpec((BLK, 1), lambda i: (i, 0)),
            pl.BlockSpec((BLK, 1), lambda i: (i, 0)),
            pl.BlockSpec((64, 128), lambda i: (0, 0)),
        ],
        out_specs=pl.BlockSpec((BLK, 128), lambda i: (i, 0)),
        out_shape=jax.ShapeDtypeStruct((NP, 128), jnp.float32),
    )(p0, p1, e0, e1, W1t)


_SC_MESH = plsc.VectorSubcoreMesh(core_axis_name="c", subcore_axis_name="s",
                                  num_cores=NC, num_subcores=NS)
_SC_PARAMS = pltpu.CompilerParams(use_tc_tiling_on_sc=False,
                                  needs_layout_passes=False)


def _wid():
    return lax.axis_index("s") * NC + lax.axis_index("c")


def _tile_max(ref, scratch16):
    def body(i, m):
        return jnp.maximum(m, ref[pl.ds(i * 16, 16)])
    m16 = lax.fori_loop(0, NP // 16, body, jnp.full((16,), -jnp.inf, jnp.float32))
    lanes = lax.iota(jnp.int32, 16)
    for step in (8, 4, 2, 1):
        scratch16[pl.ds(0, 16)] = m16
        idx = jnp.bitwise_and(lanes + step, 15)
        m16 = jnp.maximum(m16, plsc.load_gather(scratch16, [idx]))
    return m16


def _sc_pass_body(emit_aux, *refs):
    if emit_aux:
        (x_hbm, ei_hbm, as_hbm, ad_hbm, h_hbm, esum_hbm,
         as_v, ad_v, sd4, ex_v, rows2, h_sh, esum_sh,
         sem_i, sem_g, sem_s) = refs
    else:
        (x_hbm, ei_hbm, as_hbm, ad_hbm, h_hbm,
         as_v, ad_v, sd4, ex_v, rows2, h_sh,
         sem_i, sem_g, sem_s) = refs
    c = lax.axis_index("c")
    s = lax.axis_index("s")
    wid = _wid()
    pltpu.sync_copy(as_hbm, as_v)
    pltpu.sync_copy(ad_hbm, ad_v)
    def zr(i, _):
        for k in range(4):
            rows2[0, i, pl.ds(k * 16, 16)] = jnp.zeros((16,), jnp.float32)
        return 0
    lax.fori_loop(0, CHUNK, zr, 0)
    for r in range(NSLC // CHUNK):
        pltpu.sync_copy(rows2.at[0],
                        h_sh.at[pl.ds(s * NSLC + r * CHUNK, CHUNK)])
    if emit_aux:
        def ze(i, _):
            ex_v[pl.ds(i * 16, 16)] = jnp.zeros((16,), jnp.float32)
            return 0
        lax.fori_loop(0, CHUNK // 16, ze, 0)
        for r in range(NSLC // CHUNK):
            pltpu.sync_copy(ex_v, esum_sh.at[pl.ds(s * NSLC + r * CHUNK, CHUNK)])
    shift_raw = _tile_max(as_v, ex_v) + _tile_max(ad_v, ex_v)
    shift = jnp.where(shift_raw >= 0, shift_raw, 0.2 * shift_raw)
    plsc.subcore_barrier()

    rcnt = jnp.clip(RROWS - wid * TROWS, 0, TROWS)
    base_row = wid * TROWS
    pltpu.async_copy(ei_hbm.at[base_row], sd4.at[0], sem_i)

    def chunk(j, _):
        p4 = jnp.bitwise_and(j, 3)
        p2 = jnp.bitwise_and(j, 1)
        row = base_row + j
        pltpu.make_async_copy(ei_hbm.at[row], sd4.at[p4], sem_i).wait()

        @pl.when(j >= 2)
        def _():
            pltpu.make_async_copy(rows2.at[p2], h_sh.at[sd4.at[p4, 1]],
                                  sem_s).wait()
        gcp = pltpu.async_copy(x_hbm.at[sd4.at[p4, 0]], rows2.at[p2], sem_g)

        @pl.when(j + 1 < rcnt)
        def _():
            pltpu.async_copy(ei_hbm.at[row + 1],
                             sd4.at[jnp.bitwise_and(j + 1, 3)], sem_i)
        for k in range(CHUNK // 16):
            si = sd4[p4, 0, pl.ds(k * 16, 16)]
            di = sd4[p4, 1, pl.ds(k * 16, 16)]
            e = plsc.load_gather(as_v, [si]) + plsc.load_gather(ad_v, [di])
            e = jnp.where(e >= 0, e, 0.2 * e) - shift
            ex_v[pl.ds(k * 16, 16)] = jnp.exp(e)
        if emit_aux:
            pltpu.sync_copy(ex_v, esum_sh.at[sd4.at[p4, 1]], add=True)
        gcp.wait()

        @plsc.parallel_loop(0, CHUNK, step=1, unroll=4)
        def _(e4):
            ab = plsc.load_gather(ex_v, [jnp.full((16,), e4, jnp.int32)])
            for k in range(4):
                rows2[p2, e4, pl.ds(k * 16, 16)] = (
                    rows2[p2, e4, pl.ds(k * 16, 16)] * ab)

        pltpu.async_copy(rows2.at[p2], h_sh.at[sd4.at[p4, 1]], sem_s,
                         add=True)
        return 0

    lax.fori_loop(0, rcnt, chunk, 0)
    for d in range(2):
        pltpu.make_async_copy(rows2.at[d], h_sh.at[sd4.at[d, 1]],
                              sem_s).wait()
    plsc.subcore_barrier()
    pltpu.sync_copy(h_sh.at[pl.ds(s * NSLC, NSLC)],
                    h_hbm.at[pl.ds(c * NP + s * NSLC, NSLC)])
    if emit_aux:
        pltpu.sync_copy(esum_sh.at[pl.ds(s * NSLC, NSLC)],
                        esum_hbm.at[pl.ds(c * NP + s * NSLC, NSLC)])


_COMMON_SCRATCH = [
    pltpu.VMEM((NP,), jnp.float32),
    pltpu.VMEM((NP,), jnp.float32),
    pltpu.VMEM((4, 2, CHUNK), jnp.int32),
    pltpu.VMEM((CHUNK,), jnp.float32),
    pltpu.VMEM((2, CHUNK, 64), jnp.float32),
    pltpu.VMEM_SHARED((NP, 64), jnp.float32),
]

_sc_pass1 = pl.kernel(
    functools.partial(_sc_pass_body, True),
    out_type=[
        jax.ShapeDtypeStruct((NC * NP, 64), jnp.float32),
        jax.ShapeDtypeStruct((NC * NP,), jnp.float32),
    ],
    mesh=_SC_MESH,
    compiler_params=_SC_PARAMS,
    scratch_types=_COMMON_SCRATCH + [
        pltpu.VMEM_SHARED((NP,), jnp.float32),
        pltpu.SemaphoreType.DMA,
        pltpu.SemaphoreType.DMA,
        pltpu.SemaphoreType.DMA,
    ],
)

_sc_pass2 = pl.kernel(
    functools.partial(_sc_pass_body, False),
    out_type=jax.ShapeDtypeStruct((NC * NP, 64), jnp.float32),
    mesh=_SC_MESH,
    compiler_params=_SC_PARAMS,
    scratch_types=_COMMON_SCRATCH + [
        pltpu.SemaphoreType.DMA,
        pltpu.SemaphoreType.DMA,
        pltpu.SemaphoreType.DMA,
    ],
)


def kernel(features, edge_index, W1, att_src1, att_dst1, W2, gene_att, Wpred, bpred):
    src = edge_index[0].astype(jnp.int32)
    dst = edge_index[1].astype(jnp.int32)
    nrt = EP // CHUNK
    ei3 = jnp.concatenate(
        [jnp.pad(src, (0, EP - E)).reshape(nrt, 1, CHUNK),
         jnp.pad(dst, (0, EP - E)).reshape(nrt, 1, CHUNK)], axis=1)
    features_p = jnp.pad(features, ((0, NP - N), (0, 0)))
    av = jnp.stack([att_src1, att_dst1], axis=1)

    x1p, asd = _tc1(features_p, gene_att, W1, av)
    a_s = asd[:, 0]
    a_d = asd[:, 1]

    h1parts, esumP = _sc_pass1(x1p, ei3, a_s, a_d)
    e0 = esumP[:NP, None]
    e1 = esumP[NP:, None]
    h2p, x3p, lpp = _tc2(h1parts[:NP], h1parts[NP:], e0, e1,
                         W2, W2.T, Wpred, bpred[None, :])
    h3parts = _sc_pass2(x3p, ei3, a_s, a_d)
    h4p = _tc3(h3parts[:NP], h3parts[NP:], e0, e1, W1.T)

    return (h2p[:N], h4p[:N], lpp[:N])


# speedup vs baseline: 27.0871x; 1.3154x over previous
import functools

import jax
import jax.numpy as jnp
from jax import lax
from jax.experimental import pallas as pl
from jax.experimental.pallas import tpu as pltpu
from jax.experimental.pallas import tpu_sc as plsc

N = 10000
NP = 10240
E = 320000
BLK = 128
GRID = NP // BLK

NC = 2
NS = 16
NW = NC * NS
CHUNK = 128
RROWS = E // CHUNK
TROWS = 80
EP = NW * TROWS * CHUNK
NSLC = NP // NS


def _elu(x):
    return jnp.where(x > 0, x, jnp.exp(jnp.minimum(x, 0.0)) - 1.0)


def _tc1_body(f_ref, g_ref, w1_ref, av_ref, x1_ref, as_ref, ad_ref):
    att = f_ref[...] * g_ref[...]
    x1 = jnp.dot(att, w1_ref[...], preferred_element_type=jnp.float32,
                 precision=lax.Precision.HIGHEST)
    x1_ref[...] = x1
    asd = jnp.dot(x1, av_ref[...], preferred_element_type=jnp.float32,
                  precision=lax.Precision.HIGHEST)
    as_ref[...] = asd[:, 0:1]
    ad_ref[...] = asd[:, 1:2]


BLKR = 1280
GRIDR = NP // BLKR


def _tc1(features_p, gene_att, W1, av):
    return pl.pallas_call(
        _tc1_body,
        grid=(GRIDR,),
        in_specs=[
            pl.BlockSpec((BLKR, 128), lambda i: (i, 0)),
            pl.BlockSpec((1, 128), lambda i: (0, 0)),
            pl.BlockSpec((128, 64), lambda i: (0, 0)),
            pl.BlockSpec((64, 2), lambda i: (0, 0)),
        ],
        out_specs=[
            pl.BlockSpec((BLKR, 64), lambda i: (i, 0)),
            pl.BlockSpec((BLKR, 1), lambda i: (i, 0)),
            pl.BlockSpec((BLKR, 1), lambda i: (i, 0)),
        ],
        out_shape=[
            jax.ShapeDtypeStruct((NP, 64), jnp.float32),
            jax.ShapeDtypeStruct((NP, 1), jnp.float32),
            jax.ShapeDtypeStruct((NP, 1), jnp.float32),
        ],
    )(features_p, gene_att, W1, av)


def _tc2_body(p0_ref, p1_ref, e0_ref, e1_ref, w2_ref, w2t_ref, wp_ref, bp_ref,
              h2_ref, x3_ref, lp_ref):
    den = e0_ref[...] + e1_ref[...] + 1e-16
    h1 = _elu((p0_ref[...] + p1_ref[...]) / den)
    h2 = jnp.dot(h1, w2_ref[...], preferred_element_type=jnp.float32,
                 precision=lax.Precision.HIGHEST)
    h2_ref[...] = h2
    x3_ref[...] = jnp.dot(h2, w2t_ref[...], preferred_element_type=jnp.float32,
                          precision=lax.Precision.HIGHEST)
    pred = jnp.dot(h2, wp_ref[...], preferred_element_type=jnp.float32,
                   precision=lax.Precision.HIGHEST) + bp_ref[...]
    m = jnp.max(pred, axis=-1, keepdims=True)
    lse = jnp.log(jnp.sum(jnp.exp(pred - m), axis=-1, keepdims=True)) + m
    lp_ref[...] = pred - lse


def _tc2(p0, p1, e0, e1, W2, W2t, Wpred, bpred2):
    return pl.pallas_call(
        _tc2_body,
        grid=(GRIDR,),
        in_specs=[
            pl.BlockSpec((BLKR, 64), lambda i: (i, 0)),
            pl.BlockSpec((BLKR, 64), lambda i: (i, 0)),
            pl.BlockSpec((BLKR, 1), lambda i: (i, 0)),
            pl.BlockSpec((BLKR, 1), lambda i: (i, 0)),
            pl.BlockSpec((64, 32), lambda i: (0, 0)),
            pl.BlockSpec((32, 64), lambda i: (0, 0)),
            pl.BlockSpec((32, 16), lambda i: (0, 0)),
            pl.BlockSpec((1, 16), lambda i: (0, 0)),
        ],
        out_specs=[
            pl.BlockSpec((BLKR, 32), lambda i: (i, 0)),
            pl.BlockSpec((BLKR, 64), lambda i: (i, 0)),
            pl.BlockSpec((BLKR, 16), lambda i: (i, 0)),
        ],
        out_shape=[
            jax.ShapeDtypeStruct((N, 32), jnp.float32),
            jax.ShapeDtypeStruct((NP, 64), jnp.float32),
            jax.ShapeDtypeStruct((N, 16), jnp.float32),
        ],
    )(p0, p1, e0, e1, W2, W2t, Wpred, bpred2)


def _tc3_body(p0_ref, p1_ref, e0_ref, e1_ref, w1t_ref, h4_ref):
    den = e0_ref[...] + e1_ref[...] + 1e-16
    h3 = _elu((p0_ref[...] + p1_ref[...]) / den)
    h4 = jnp.dot(h3, w1t_ref[...], preferred_element_type=jnp.float32,
                 precision=lax.Precision.HIGHEST)
    h4_ref[...] = h4


def _tc3(p0, p1, e0, e1, W1t):
    return pl.pallas_call(
        _tc3_body,
        grid=(GRIDR,),
        in_specs=[
            pl.BlockSpec((BLKR, 64), lambda i: (i, 0)),
            pl.BlockSpec((BLKR, 64), lambda i: (i, 0)),
            pl.BlockSpec((BLKR, 1), lambda i: (i, 0)),
            pl.BlockSpec((BLKR, 1), lambda i: (i, 0)),
            pl.BlockSpec((64, 128), lambda i: (0, 0)),
        ],
        out_specs=pl.BlockSpec((BLKR, 128), lambda i: (i, 0)),
        out_shape=jax.ShapeDtypeStruct((N, 128), jnp.float32),
    )(p0, p1, e0, e1, W1t)


_SC_MESH = plsc.VectorSubcoreMesh(core_axis_name="c", subcore_axis_name="s",
                                  num_cores=NC, num_subcores=NS)
_SC_PARAMS = pltpu.CompilerParams(use_tc_tiling_on_sc=False,
                                  needs_layout_passes=False)


def _wid():
    return lax.axis_index("s") * NC + lax.axis_index("c")


def _tile_max(ref, scratch16):
    def body(i, m):
        return jnp.maximum(m, ref[pl.ds(i * 16, 16)])
    m16 = lax.fori_loop(0, NP // 16, body, jnp.full((16,), -jnp.inf, jnp.float32))
    lanes = lax.iota(jnp.int32, 16)
    for step in (8, 4, 2, 1):
        scratch16[pl.ds(0, 16)] = m16
        idx = jnp.bitwise_and(lanes + step, 15)
        m16 = jnp.maximum(m16, plsc.load_gather(scratch16, [idx]))
    return m16


def _sc_pass_body(emit_aux, *refs):
    if emit_aux:
        (x_hbm, src_hbm, dst_hbm, as_hbm, ad_hbm, h_hbm, esum_hbm,
         as_v, ad_v, sd4, ex_v, rows2, h_sh, esum_sh,
         sem_i, sem_g, sem_s) = refs
    else:
        (x_hbm, src_hbm, dst_hbm, as_hbm, ad_hbm, h_hbm,
         as_v, ad_v, sd4, ex_v, rows2, h_sh,
         sem_i, sem_g, sem_s) = refs
    c = lax.axis_index("c")
    s = lax.axis_index("s")
    wid = _wid()
    pltpu.sync_copy(as_hbm, as_v)
    pltpu.sync_copy(ad_hbm, ad_v)
    def zr(i, _):
        for k in range(4):
            rows2[0, i, pl.ds(k * 16, 16)] = jnp.zeros((16,), jnp.float32)
        return 0
    lax.fori_loop(0, CHUNK, zr, 0)
    for r in range(NSLC // CHUNK):
        pltpu.sync_copy(rows2.at[0],
                        h_sh.at[pl.ds(s * NSLC + r * CHUNK, CHUNK)])
    if emit_aux:
        def ze(i, _):
            ex_v[pl.ds(i * 16, 16)] = jnp.zeros((16,), jnp.float32)
            return 0
        lax.fori_loop(0, CHUNK // 16, ze, 0)
        for r in range(NSLC // CHUNK):
            pltpu.sync_copy(ex_v, esum_sh.at[pl.ds(s * NSLC + r * CHUNK, CHUNK)])
    shift_raw = _tile_max(as_v, ex_v) + _tile_max(ad_v, ex_v)
    shift = jnp.where(shift_raw >= 0, shift_raw, 0.2 * shift_raw)
    plsc.subcore_barrier()

    rcnt = jnp.clip(RROWS - wid * TROWS, 0, TROWS)
    base_row = wid * TROWS
    pltpu.async_copy(src_hbm.at[pl.ds(base_row * CHUNK, CHUNK)],
                     sd4.at[0, 0], sem_i)
    pltpu.async_copy(dst_hbm.at[pl.ds(base_row * CHUNK, CHUNK)],
                     sd4.at[0, 1], sem_i)

    def chunk(j, _):
        p4 = jnp.bitwise_and(j, 3)
        p2 = jnp.bitwise_and(j, 1)
        row = base_row + j
        pltpu.make_async_copy(src_hbm.at[pl.ds(row * CHUNK, CHUNK)],
                              sd4.at[p4, 0], sem_i).wait()
        pltpu.make_async_copy(dst_hbm.at[pl.ds(row * CHUNK, CHUNK)],
                              sd4.at[p4, 1], sem_i).wait()

        @pl.when(j >= 2)
        def _():
            pltpu.make_async_copy(rows2.at[p2], h_sh.at[sd4.at[p4, 1]],
                                  sem_s).wait()
        gcp = pltpu.async_copy(x_hbm.at[sd4.at[p4, 0]], rows2.at[p2], sem_g)

        @pl.when(j + 1 < rcnt)
        def _():
            pn4 = jnp.bitwise_and(j + 1, 3)
            pltpu.async_copy(src_hbm.at[pl.ds((row + 1) * CHUNK, CHUNK)],
                             sd4.at[pn4, 0], sem_i)
            pltpu.async_copy(dst_hbm.at[pl.ds((row + 1) * CHUNK, CHUNK)],
                             sd4.at[pn4, 1], sem_i)
        for k in range(CHUNK // 16):
            si = sd4[p4, 0, pl.ds(k * 16, 16)]
            di = sd4[p4, 1, pl.ds(k * 16, 16)]
            e = plsc.load_gather(as_v, [si]) + plsc.load_gather(ad_v, [di])
            e = jnp.where(e >= 0, e, 0.2 * e) - shift
            ex_v[pl.ds(k * 16, 16)] = jnp.exp(e)
        if emit_aux:
            pltpu.sync_copy(ex_v, esum_sh.at[sd4.at[p4, 1]], add=True)
        gcp.wait()

        @plsc.parallel_loop(0, CHUNK, step=1, unroll=4)
        def _(e4):
            ab = plsc.load_gather(ex_v, [jnp.full((16,), e4, jnp.int32)])
            for k in range(4):
                rows2[p2, e4, pl.ds(k * 16, 16)] = (
                    rows2[p2, e4, pl.ds(k * 16, 16)] * ab)

        pltpu.async_copy(rows2.at[p2], h_sh.at[sd4.at[p4, 1]], sem_s,
                         add=True)
        return 0

    lax.fori_loop(0, rcnt, chunk, 0)
    for d in range(2):
        pltpu.make_async_copy(rows2.at[d], h_sh.at[sd4.at[d, 1]],
                              sem_s).wait()
    plsc.subcore_barrier()
    pltpu.sync_copy(h_sh.at[pl.ds(s * NSLC, NSLC)],
                    h_hbm.at[pl.ds(c * NP + s * NSLC, NSLC)])
    if emit_aux:
        pltpu.sync_copy(esum_sh.at[pl.ds(s * NSLC, NSLC)],
                        esum_hbm.at[pl.ds(c * NP + s * NSLC, NSLC)])


_COMMON_SCRATCH = [
    pltpu.VMEM((NP,), jnp.float32),
    pltpu.VMEM((NP,), jnp.float32),
    pltpu.VMEM((4, 2, CHUNK), jnp.int32),
    pltpu.VMEM((CHUNK,), jnp.float32),
    pltpu.VMEM((2, CHUNK, 64), jnp.float32),
    pltpu.VMEM_SHARED((NP, 64), jnp.float32),
]

_sc_pass1 = pl.kernel(
    functools.partial(_sc_pass_body, True),
    out_type=[
        jax.ShapeDtypeStruct((NC * NP, 64), jnp.float32),
        jax.ShapeDtypeStruct((NC * NP,), jnp.float32),
    ],
    mesh=_SC_MESH,
    compiler_params=_SC_PARAMS,
    scratch_types=_COMMON_SCRATCH + [
        pltpu.VMEM_SHARED((NP,), jnp.float32),
        pltpu.SemaphoreType.DMA,
        pltpu.SemaphoreType.DMA,
        pltpu.SemaphoreType.DMA,
    ],
)

_sc_pass2 = pl.kernel(
    functools.partial(_sc_pass_body, False),
    out_type=jax.ShapeDtypeStruct((NC * NP, 64), jnp.float32),
    mesh=_SC_MESH,
    compiler_params=_SC_PARAMS,
    scratch_types=_COMMON_SCRATCH + [
        pltpu.SemaphoreType.DMA,
        pltpu.SemaphoreType.DMA,
        pltpu.SemaphoreType.DMA,
    ],
)


def kernel(features, edge_index, W1, att_src1, att_dst1, W2, gene_att, Wpred, bpred):
    srcv = edge_index[0].astype(jnp.int32)
    dstv = edge_index[1].astype(jnp.int32)
    features_p = jnp.pad(features, ((0, NP - N), (0, 0)))
    av = jnp.stack([att_src1, att_dst1], axis=1)

    x1p, as_c, ad_c = _tc1(features_p, gene_att, W1, av)
    a_s = as_c.reshape(NP)
    a_d = ad_c.reshape(NP)

    h1parts, esumP = _sc_pass1(x1p, srcv, dstv, a_s, a_d)
    e0 = esumP[:NP, None]
    e1 = esumP[NP:, None]
    h2, x3p, lp = _tc2(h1parts[:NP], h1parts[NP:], e0, e1,
                       W2, W2.T, Wpred, bpred[None, :])
    h3parts = _sc_pass2(x3p, srcv, dstv, a_s, a_d)
    h4 = _tc3(h3parts[:NP], h3parts[NP:], e0, e1, W1.T)

    return (h2, h4, lp)


# speedup vs baseline: 36.9258x; 1.3632x over previous
import functools

import jax
import jax.numpy as jnp
from jax import lax
from jax.experimental import pallas as pl
from jax.experimental.pallas import tpu as pltpu
from jax.experimental.pallas import tpu_sc as plsc

N = 10000
NP = 10240
E = 320000
NROW = NP // 128
BLKR = 1024
GRIDR = NP // BLKR
BROW = BLKR // 128

NC = 2
NS = 16
NW = NC * NS
CHUNK = 128
RROWS = E // CHUNK
TROWS = 80
NSLC = NP // NS


def _elu(x):
    return jnp.where(x > 0, x, jnp.exp(jnp.minimum(x, 0.0)) - 1.0)


def _tc1_body(f_ref, g_ref, w1_ref, av_ref, x1_ref, asd_ref):
    att = f_ref[...] * g_ref[...]
    x1 = jnp.dot(att, w1_ref[...], preferred_element_type=jnp.float32,
                 precision=lax.Precision.HIGHEST)
    x1_ref[...] = x1
    asdT = lax.dot_general(av_ref[...], x1,
                           dimension_numbers=(((0,), (1,)), ((), ())),
                           preferred_element_type=jnp.float32,
                           precision=lax.Precision.HIGHEST)
    asd_ref[...] = asdT


def _tc1(features_p, gene_att, W1, av):
    return pl.pallas_call(
        _tc1_body,
        grid=(GRIDR,),
        in_specs=[
            pl.BlockSpec((BLKR, 128), lambda i: (i, 0)),
            pl.BlockSpec((1, 128), lambda i: (0, 0)),
            pl.BlockSpec((128, 64), lambda i: (0, 0)),
            pl.BlockSpec((64, 2), lambda i: (0, 0)),
        ],
        out_specs=[
            pl.BlockSpec((BLKR, 64), lambda i: (i, 0)),
            pl.BlockSpec((2, BLKR), lambda i: (0, i)),
        ],
        out_shape=[
            jax.ShapeDtypeStruct((NP, 64), jnp.float32),
            jax.ShapeDtypeStruct((2, NP), jnp.float32),
        ],
    )(features_p, gene_att, W1, av)


def _tc2_body(p0_ref, p1_ref, e0_ref, e1_ref, w2_ref, w2t_ref, wp_ref, bp_ref,
              h2_ref, x3_ref, lp_ref):
    den = e0_ref[...] + e1_ref[...] + 1e-16
    den64 = jnp.reshape(jnp.broadcast_to(den[:, :, None], (BROW, 128, 64)),
                        (BLKR, 64))
    h1 = _elu((p0_ref[...] + p1_ref[...]) / den64)
    h2 = jnp.dot(h1, w2_ref[...], preferred_element_type=jnp.float32,
                 precision=lax.Precision.HIGHEST)
    h2_ref[...] = h2
    x3_ref[...] = jnp.dot(h2, w2t_ref[...], preferred_element_type=jnp.float32,
                          precision=lax.Precision.HIGHEST)
    pred = jnp.dot(h2, wp_ref[...], preferred_element_type=jnp.float32,
                   precision=lax.Precision.HIGHEST) + bp_ref[...]
    m = jnp.max(pred, axis=-1, keepdims=True)
    lse = jnp.log(jnp.sum(jnp.exp(pred - m), axis=-1, keepdims=True)) + m
    lp_ref[...] = pred - lse


def _tc2(hparts, esum2, W2, W2t, Wpred, bpred2):
    return pl.pallas_call(
        _tc2_body,
        grid=(GRIDR,),
        in_specs=[
            pl.BlockSpec((BLKR, 64), lambda i: (i, 0)),
            pl.BlockSpec((BLKR, 64), lambda i: (i + GRIDR, 0)),
            pl.BlockSpec((BROW, 128), lambda i: (i, 0)),
            pl.BlockSpec((BROW, 128), lambda i: (i + GRIDR, 0)),
            pl.BlockSpec((64, 32), lambda i: (0, 0)),
            pl.BlockSpec((32, 64), lambda i: (0, 0)),
            pl.BlockSpec((32, 16), lambda i: (0, 0)),
            pl.BlockSpec((1, 16), lambda i: (0, 0)),
        ],
        out_specs=[
            pl.BlockSpec((BLKR, 32), lambda i: (i, 0)),
            pl.BlockSpec((BLKR, 64), lambda i: (i, 0)),
            pl.BlockSpec((BLKR, 16), lambda i: (i, 0)),
        ],
        out_shape=[
            jax.ShapeDtypeStruct((N, 32), jnp.float32),
            jax.ShapeDtypeStruct((NP, 64), jnp.float32),
            jax.ShapeDtypeStruct((N, 16), jnp.float32),
        ],
    )(hparts, hparts, esum2, esum2, W2, W2t, Wpred, bpred2)


def _tc3_body(p0_ref, p1_ref, e0_ref, e1_ref, w1t_ref, h4_ref):
    den = e0_ref[...] + e1_ref[...] + 1e-16
    den64 = jnp.reshape(jnp.broadcast_to(den[:, :, None], (BROW, 128, 64)),
                        (BLKR, 64))
    h3 = _elu((p0_ref[...] + p1_ref[...]) / den64)
    h4_ref[...] = jnp.dot(h3, w1t_ref[...], preferred_element_type=jnp.float32,
                          precision=lax.Precision.HIGHEST)


def _tc3(hparts, esum2, W1t):
    return pl.pallas_call(
        _tc3_body,
        grid=(GRIDR,),
        in_specs=[
            pl.BlockSpec((BLKR, 64), lambda i: (i, 0)),
            pl.BlockSpec((BLKR, 64), lambda i: (i + GRIDR, 0)),
            pl.BlockSpec((BROW, 128), lambda i: (i, 0)),
            pl.BlockSpec((BROW, 128), lambda i: (i + GRIDR, 0)),
            pl.BlockSpec((64, 128), lambda i: (0, 0)),
        ],
        out_specs=pl.BlockSpec((BLKR, 128), lambda i: (i, 0)),
        out_shape=jax.ShapeDtypeStruct((N, 128), jnp.float32),
    )(hparts, hparts, esum2, esum2, W1t)


_SC_MESH = plsc.VectorSubcoreMesh(core_axis_name="c", subcore_axis_name="s",
                                  num_cores=NC, num_subcores=NS)
_SC_PARAMS = pltpu.CompilerParams(use_tc_tiling_on_sc=False,
                                  needs_layout_passes=False)


def _wid():
    return lax.axis_index("s") * NC + lax.axis_index("c")


def _tile_max(ref1, scratch16):
    def body(i, m):
        return jnp.maximum(m, ref1[pl.ds(i * 16, 16)])
    m16 = lax.fori_loop(0, NP // 16, body, jnp.full((16,), -jnp.inf, jnp.float32))
    lanes = lax.iota(jnp.int32, 16)
    for step in (8, 4, 2, 1):
        scratch16[pl.ds(0, 16)] = m16
        idx = jnp.bitwise_and(lanes + step, 15)
        m16 = jnp.maximum(m16, plsc.load_gather(scratch16, [idx]))
    return m16


def _gather_tab(tab1, idx16):
    return plsc.load_gather(tab1, [idx16])


def _sc_pass_body(emit_aux, *refs):
    if emit_aux:
        (x_hbm, sd_hbm, as_hbm, ad_hbm, h_hbm, esum_hbm,
         as_v, ad_v, sd4, ex_v, rows2, h_sh, esum_sh,
         sem_i, sem_g, sem_s) = refs
    else:
        (x_hbm, sd_hbm, as_hbm, ad_hbm, h_hbm,
         as_v, ad_v, sd4, ex_v, rows2, h_sh,
         sem_i, sem_g, sem_s) = refs
    c = lax.axis_index("c")
    s = lax.axis_index("s")
    wid = _wid()
    pltpu.sync_copy(as_hbm, as_v)
    pltpu.sync_copy(ad_hbm, ad_v)
    def zr(i, _):
        for k in range(4):
            rows2[0, i, pl.ds(k * 16, 16)] = jnp.zeros((16,), jnp.float32)
        return 0
    lax.fori_loop(0, CHUNK, zr, 0)
    for r in range(NSLC // CHUNK):
        pltpu.sync_copy(rows2.at[0],
                        h_sh.at[pl.ds(s * NSLC + r * CHUNK, CHUNK)])
    if emit_aux:
        def ze(i, _):
            ex_v[pl.ds(i * 16, 16)] = jnp.zeros((16,), jnp.float32)
            return 0
        lax.fori_loop(0, CHUNK // 16, ze, 0)
        for r in range(NSLC // CHUNK):
            pltpu.sync_copy(ex_v, esum_sh.at[pl.ds(s * NSLC + r * CHUNK, CHUNK)])
    shift_raw = _tile_max(as_v, ex_v) + _tile_max(ad_v, ex_v)
    shift = jnp.where(shift_raw >= 0, shift_raw, 0.2 * shift_raw)
    plsc.subcore_barrier()

    rcnt = jnp.clip(RROWS - wid * TROWS, 0, TROWS)
    base_row = wid * TROWS

    def idx_issue(r, slot):
        pltpu.async_copy(sd_hbm.at[pl.ds(r * CHUNK, CHUNK)],
                         sd4.at[slot, 0], sem_i)
        pltpu.async_copy(sd_hbm.at[pl.ds(E + r * CHUNK, CHUNK)],
                         sd4.at[slot, 1], sem_i)

    def idx_wait(r, slot):
        pltpu.make_async_copy(sd_hbm.at[pl.ds(r * CHUNK, CHUNK)],
                              sd4.at[slot, 0], sem_i).wait()
        pltpu.make_async_copy(sd_hbm.at[pl.ds(E + r * CHUNK, CHUNK)],
                              sd4.at[slot, 1], sem_i).wait()

    def gather_issue(r, slot, p2):
        pltpu.async_copy(x_hbm.at[sd4.at[slot, 0]], rows2.at[p2], sem_g)

    def scat_wait(p2, slot):
        pltpu.make_async_copy(rows2.at[p2], h_sh.at[sd4.at[slot, 1]],
                              sem_s).wait()

    idx_issue(base_row, 0)
    idx_wait(base_row, 0)
    gather_issue(base_row, 0, 0)
    idx_issue(base_row + 1, 1)

    def chunk(j, _):
        p4 = jnp.bitwise_and(j, 3)
        p2 = jnp.bitwise_and(j, 1)
        row = base_row + j
        for k in range(CHUNK // 16):
            si = sd4[p4, 0, pl.ds(k * 16, 16)]
            di = sd4[p4, 1, pl.ds(k * 16, 16)]
            e = _gather_tab(as_v, si) + _gather_tab(ad_v, di)
            e = jnp.where(e >= 0, e, 0.2 * e) - shift
            ex_v[pl.ds(k * 16, 16)] = jnp.exp(e)
        if emit_aux:
            pltpu.sync_copy(ex_v, esum_sh.at[sd4.at[p4, 1]], add=True)

        @pl.when(j + 1 < rcnt)
        def _():
            n4 = jnp.bitwise_and(j + 1, 3)
            n2 = jnp.bitwise_and(j + 1, 1)
            idx_wait(row + 1, n4)

            @pl.when(j >= 1)
            def _():
                scat_wait(n2, jnp.bitwise_and(j - 1, 3))
            gather_issue(row + 1, n4, n2)

            @pl.when(j + 2 < rcnt)
            def _():
                idx_issue(row + 2, jnp.bitwise_and(j + 2, 3))

        pltpu.make_async_copy(x_hbm.at[sd4.at[p4, 0]], rows2.at[p2],
                              sem_g).wait()

        @plsc.parallel_loop(0, CHUNK, step=1, unroll=4)
        def _(e4):
            ab = plsc.load_gather(ex_v, [jnp.full((16,), e4, jnp.int32)])
            for k in range(4):
                rows2[p2, e4, pl.ds(k * 16, 16)] = (
                    rows2[p2, e4, pl.ds(k * 16, 16)] * ab)

        pltpu.async_copy(rows2.at[p2], h_sh.at[sd4.at[p4, 1]], sem_s,
                         add=True)
        return 0

    lax.fori_loop(0, rcnt, chunk, 0)
    for d in range(2):
        pltpu.make_async_copy(rows2.at[d], h_sh.at[sd4.at[d, 1]],
                              sem_s).wait()
    plsc.subcore_barrier()
    pltpu.sync_copy(h_sh.at[pl.ds(s * NSLC, NSLC)],
                    h_hbm.at[pl.ds(c * NP + s * NSLC, NSLC)])
    if emit_aux:
        pltpu.sync_copy(esum_sh.at[pl.ds(s * NSLC, NSLC)],
                        esum_hbm.at[pl.ds(c * NP + s * NSLC, NSLC)])


_COMMON_SCRATCH = [
    pltpu.VMEM((NP,), jnp.float32),
    pltpu.VMEM((NP,), jnp.float32),
    pltpu.VMEM((4, 2, CHUNK), jnp.int32),
    pltpu.VMEM((CHUNK,), jnp.float32),
    pltpu.VMEM((2, CHUNK, 64), jnp.float32),
    pltpu.VMEM_SHARED((NP, 64), jnp.float32),
]

_sc_pass1 = pl.kernel(
    functools.partial(_sc_pass_body, True),
    out_type=[
        jax.ShapeDtypeStruct((NC * NP, 64), jnp.float32),
        jax.ShapeDtypeStruct((NC * NP,), jnp.float32),
    ],
    mesh=_SC_MESH,
    compiler_params=_SC_PARAMS,
    scratch_types=_COMMON_SCRATCH + [
        pltpu.VMEM_SHARED((NP,), jnp.float32),
        pltpu.SemaphoreType.DMA,
        pltpu.SemaphoreType.DMA,
        pltpu.SemaphoreType.DMA,
    ],
)

_sc_pass2 = pl.kernel(
    functools.partial(_sc_pass_body, False),
    out_type=jax.ShapeDtypeStruct((NC * NP, 64), jnp.float32),
    mesh=_SC_MESH,
    compiler_params=_SC_PARAMS,
    scratch_types=_COMMON_SCRATCH + [
        pltpu.SemaphoreType.DMA,
        pltpu.SemaphoreType.DMA,
        pltpu.SemaphoreType.DMA,
    ],
)


def kernel(features, edge_index, W1, att_src1, att_dst1, W2, gene_att, Wpred, bpred):
    sd = edge_index.astype(jnp.int32).reshape(2 * E)
    features_p = jnp.pad(features, ((0, NP - N), (0, 0)))
    av = jnp.stack([att_src1, att_dst1], axis=1)

    x1p, asdT = _tc1(features_p, gene_att, W1, av)
    a_s = asdT[0]
    a_d = asdT[1]

    h1parts, esumP = _sc_pass1(x1p, sd, a_s, a_d)
    esum2 = esumP.reshape(NC * NROW, 128)
    h2, x3p, lp = _tc2(h1parts, esum2, W2, W2.T, Wpred, bpred[None, :])
    h3parts = _sc_pass2(x3p, sd, a_s, a_d)
    h4 = _tc3(h3parts, esum2, W1.T)

    return (h2, h4, lp)


# speedup vs baseline: 38.5480x; 1.0439x over previous
import functools

import jax
import jax.numpy as jnp
from jax import lax
from jax.experimental import pallas as pl
from jax.experimental.pallas import tpu as pltpu
from jax.experimental.pallas import tpu_sc as plsc

N = 10000
NP = 10240
E = 320000
NROW = NP // 128
BLKR = 2048
GRIDR = NP // BLKR
BROW = BLKR // 128

NC = 2
NS = 16
NW = NC * NS
CHUNK = 128
RROWS = E // CHUNK
TROWS = 80
NSLC = NP // NS


def _elu(x):
    return jnp.where(x > 0, x, jnp.exp(jnp.minimum(x, 0.0)) - 1.0)


def _tc1_body(f_ref, g_ref, w1_ref, av_ref, x1_ref, asd_ref):
    att = f_ref[...] * g_ref[...]
    x1 = jnp.dot(att, w1_ref[...], preferred_element_type=jnp.float32,
                 precision=lax.Precision.HIGHEST)
    x1_ref[...] = x1
    asdT = lax.dot_general(av_ref[...], x1,
                           dimension_numbers=(((0,), (1,)), ((), ())),
                           preferred_element_type=jnp.float32,
                           precision=lax.Precision.HIGHEST)
    asd_ref[...] = asdT


def _tc1(features_p, gene_att, W1, av):
    return pl.pallas_call(
        _tc1_body,
        grid=(GRIDR,),
        in_specs=[
            pl.BlockSpec((BLKR, 128), lambda i: (i, 0)),
            pl.BlockSpec((1, 128), lambda i: (0, 0)),
            pl.BlockSpec((128, 64), lambda i: (0, 0)),
            pl.BlockSpec((64, 2), lambda i: (0, 0)),
        ],
        out_specs=[
            pl.BlockSpec((BLKR, 64), lambda i: (i, 0)),
            pl.BlockSpec((2, BLKR), lambda i: (0, i)),
        ],
        out_shape=[
            jax.ShapeDtypeStruct((NP, 64), jnp.float32),
            jax.ShapeDtypeStruct((2, NP), jnp.float32),
        ],
    )(features_p, gene_att, W1, av)


def _tc2_body(p0_ref, p1_ref, e0_ref, e1_ref, w2_ref, w2t_ref, wp_ref, bp_ref,
              h2_ref, x3_ref, lp_ref):
    den = e0_ref[...] + e1_ref[...] + 1e-16
    den64 = jnp.reshape(jnp.broadcast_to(den[:, :, None], (BROW, 128, 64)),
                        (BLKR, 64))
    h1 = _elu((p0_ref[...] + p1_ref[...]) / den64)
    h2_ref[...] = jnp.dot(h1, w2_ref[...], preferred_element_type=jnp.float32,
                          precision=lax.Precision.HIGHEST)
    x3_ref[...] = jnp.dot(h1, w2t_ref[...], preferred_element_type=jnp.float32,
                          precision=lax.Precision.HIGHEST)
    pred = jnp.dot(h1, wp_ref[...], preferred_element_type=jnp.float32,
                   precision=lax.Precision.HIGHEST) + bp_ref[...]
    m = jnp.max(pred, axis=-1, keepdims=True)
    lse = jnp.log(jnp.sum(jnp.exp(pred - m), axis=-1, keepdims=True)) + m
    lp_ref[...] = pred - lse


def _tc2(hparts, esum2, W2, W2t, Wpred, bpred2):
    return pl.pallas_call(
        _tc2_body,
        grid=(GRIDR,),
        in_specs=[
            pl.BlockSpec((BLKR, 64), lambda i: (i, 0)),
            pl.BlockSpec((BLKR, 64), lambda i: (i + GRIDR, 0)),
            pl.BlockSpec((BROW, 128), lambda i: (i, 0)),
            pl.BlockSpec((BROW, 128), lambda i: (i + GRIDR, 0)),
            pl.BlockSpec((64, 32), lambda i: (0, 0)),
            pl.BlockSpec((64, 64), lambda i: (0, 0)),
            pl.BlockSpec((64, 16), lambda i: (0, 0)),
            pl.BlockSpec((1, 16), lambda i: (0, 0)),
        ],
        out_specs=[
            pl.BlockSpec((BLKR, 32), lambda i: (i, 0)),
            pl.BlockSpec((BLKR, 64), lambda i: (i, 0)),
            pl.BlockSpec((BLKR, 16), lambda i: (i, 0)),
        ],
        out_shape=[
            jax.ShapeDtypeStruct((N, 32), jnp.float32),
            jax.ShapeDtypeStruct((NP, 64), jnp.float32),
            jax.ShapeDtypeStruct((N, 16), jnp.float32),
        ],
    )(hparts, hparts, esum2, esum2, W2, W2t, Wpred, bpred2)


def _tc3_body(p0_ref, p1_ref, e0_ref, e1_ref, w1t_ref, h4_ref):
    den = e0_ref[...] + e1_ref[...] + 1e-16
    den64 = jnp.reshape(jnp.broadcast_to(den[:, :, None], (BROW, 128, 64)),
                        (BLKR, 64))
    h3 = _elu((p0_ref[...] + p1_ref[...]) / den64)
    h4_ref[...] = jnp.dot(h3, w1t_ref[...], preferred_element_type=jnp.float32,
                          precision=lax.Precision.HIGHEST)


def _tc3(hparts, esum2, W1t):
    return pl.pallas_call(
        _tc3_body,
        grid=(GRIDR,),
        in_specs=[
            pl.BlockSpec((BLKR, 64), lambda i: (i, 0)),
            pl.BlockSpec((BLKR, 64), lambda i: (i + GRIDR, 0)),
            pl.BlockSpec((BROW, 128), lambda i: (i, 0)),
            pl.BlockSpec((BROW, 128), lambda i: (i + GRIDR, 0)),
            pl.BlockSpec((64, 128), lambda i: (0, 0)),
        ],
        out_specs=pl.BlockSpec((BLKR, 128), lambda i: (i, 0)),
        out_shape=jax.ShapeDtypeStruct((N, 128), jnp.float32),
    )(hparts, hparts, esum2, esum2, W1t)


_SC_MESH = plsc.VectorSubcoreMesh(core_axis_name="c", subcore_axis_name="s",
                                  num_cores=NC, num_subcores=NS)
_SC_PARAMS = pltpu.CompilerParams(use_tc_tiling_on_sc=False,
                                  needs_layout_passes=False)


def _wid():
    return lax.axis_index("s") * NC + lax.axis_index("c")


def _tile_max(ref1, scratch16):
    def body(i, m):
        return jnp.maximum(m, ref1[pl.ds(i * 16, 16)])
    m16 = lax.fori_loop(0, NP // 16, body, jnp.full((16,), -jnp.inf, jnp.float32))
    lanes = lax.iota(jnp.int32, 16)
    for step in (8, 4, 2, 1):
        scratch16[pl.ds(0, 16)] = m16
        idx = jnp.bitwise_and(lanes + step, 15)
        m16 = jnp.maximum(m16, plsc.load_gather(scratch16, [idx]))
    return m16


def _gather_tab(tab1, idx16):
    return plsc.load_gather(tab1, [idx16])


def _sc_pass_body(emit_aux, *refs):
    if emit_aux:
        (x_hbm, sd_hbm, as_hbm, ad_hbm, h_hbm, esum_hbm,
         as_v, ad_v, sd4, ex_v, rows2, h_sh, esum_sh,
         sem_i, sem_g, sem_s) = refs
    else:
        (x_hbm, sd_hbm, as_hbm, ad_hbm, h_hbm,
         as_v, ad_v, sd4, ex_v, rows2, h_sh,
         sem_i, sem_g, sem_s) = refs
    c = lax.axis_index("c")
    s = lax.axis_index("s")
    wid = _wid()
    pltpu.sync_copy(as_hbm, as_v)
    pltpu.sync_copy(ad_hbm, ad_v)
    def zr(i, _):
        for k in range(4):
            rows2[0, i, pl.ds(k * 16, 16)] = jnp.zeros((16,), jnp.float32)
        return 0
    lax.fori_loop(0, CHUNK, zr, 0)
    for r in range(NSLC // CHUNK):
        pltpu.sync_copy(rows2.at[0],
                        h_sh.at[pl.ds(s * NSLC + r * CHUNK, CHUNK)])
    if emit_aux:
        def ze(i, _):
            ex_v[pl.ds(i * 16, 16)] = jnp.zeros((16,), jnp.float32)
            return 0
        lax.fori_loop(0, CHUNK // 16, ze, 0)
        for r in range(NSLC // CHUNK):
            pltpu.sync_copy(ex_v, esum_sh.at[pl.ds(s * NSLC + r * CHUNK, CHUNK)])
    shift_raw = _tile_max(as_v, ex_v) + _tile_max(ad_v, ex_v)
    shift = jnp.where(shift_raw >= 0, shift_raw, 0.2 * shift_raw)
    plsc.subcore_barrier()

    rcnt = jnp.clip(RROWS - wid * TROWS, 0, TROWS)
    base_row = wid * TROWS

    def idx_issue(r, slot):
        pltpu.async_copy(sd_hbm.at[pl.ds(r * CHUNK, CHUNK)],
                         sd4.at[slot, 0], sem_i)
        pltpu.async_copy(sd_hbm.at[pl.ds(E + r * CHUNK, CHUNK)],
                         sd4.at[slot, 1], sem_i)

    def idx_wait(r, slot):
        pltpu.make_async_copy(sd_hbm.at[pl.ds(r * CHUNK, CHUNK)],
                              sd4.at[slot, 0], sem_i).wait()
        pltpu.make_async_copy(sd_hbm.at[pl.ds(E + r * CHUNK, CHUNK)],
                              sd4.at[slot, 1], sem_i).wait()

    def gather_issue(r, slot, p2):
        pltpu.async_copy(x_hbm.at[sd4.at[slot, 0]], rows2.at[p2], sem_g)

    def scat_wait(p2, slot):
        pltpu.make_async_copy(rows2.at[p2], h_sh.at[sd4.at[slot, 1]],
                              sem_s).wait()

    idx_issue(base_row, 0)
    idx_wait(base_row, 0)
    gather_issue(base_row, 0, 0)
    idx_issue(base_row + 1, 1)

    def chunk(j, _):
        p4 = jnp.bitwise_and(j, 3)
        p2 = jnp.bitwise_and(j, 1)
        row = base_row + j
        for k in range(CHUNK // 16):
            si = sd4[p4, 0, pl.ds(k * 16, 16)]
            di = sd4[p4, 1, pl.ds(k * 16, 16)]
            e = _gather_tab(as_v, si) + _gather_tab(ad_v, di)
            e = jnp.where(e >= 0, e, 0.2 * e) - shift
            ex_v[pl.ds(k * 16, 16)] = jnp.exp(e)
        if emit_aux:
            pltpu.sync_copy(ex_v, esum_sh.at[sd4.at[p4, 1]], add=True)

        @pl.when(j + 1 < rcnt)
        def _():
            n4 = jnp.bitwise_and(j + 1, 3)
            n2 = jnp.bitwise_and(j + 1, 1)
            idx_wait(row + 1, n4)

            @pl.when(j >= 1)
            def _():
                scat_wait(n2, jnp.bitwise_and(j - 1, 3))
            gather_issue(row + 1, n4, n2)

            @pl.when(j + 2 < rcnt)
            def _():
                idx_issue(row + 2, jnp.bitwise_and(j + 2, 3))

        pltpu.make_async_copy(x_hbm.at[sd4.at[p4, 0]], rows2.at[p2],
                              sem_g).wait()

        @plsc.parallel_loop(0, CHUNK, step=1, unroll=4)
        def _(e4):
            ab = plsc.load_gather(ex_v, [jnp.full((16,), e4, jnp.int32)])
            for k in range(4):
                rows2[p2, e4, pl.ds(k * 16, 16)] = (
                    rows2[p2, e4, pl.ds(k * 16, 16)] * ab)

        pltpu.async_copy(rows2.at[p2], h_sh.at[sd4.at[p4, 1]], sem_s,
                         add=True)
        return 0

    lax.fori_loop(0, rcnt, chunk, 0)
    for d in range(2):
        pltpu.make_async_copy(rows2.at[d], h_sh.at[sd4.at[d, 1]],
                              sem_s).wait()
    plsc.subcore_barrier()
    pltpu.sync_copy(h_sh.at[pl.ds(s * NSLC, NSLC)],
                    h_hbm.at[pl.ds(c * NP + s * NSLC, NSLC)])
    if emit_aux:
        pltpu.sync_copy(esum_sh.at[pl.ds(s * NSLC, NSLC)],
                        esum_hbm.at[pl.ds(c * NP + s * NSLC, NSLC)])


_COMMON_SCRATCH = [
    pltpu.VMEM((NP,), jnp.float32),
    pltpu.VMEM((NP,), jnp.float32),
    pltpu.VMEM((4, 2, CHUNK), jnp.int32),
    pltpu.VMEM((CHUNK,), jnp.float32),
    pltpu.VMEM((2, CHUNK, 64), jnp.float32),
    pltpu.VMEM_SHARED((NP, 64), jnp.float32),
]

_sc_pass1 = pl.kernel(
    functools.partial(_sc_pass_body, True),
    out_type=[
        jax.ShapeDtypeStruct((NC * NP, 64), jnp.float32),
        jax.ShapeDtypeStruct((NC * NP,), jnp.float32),
    ],
    mesh=_SC_MESH,
    compiler_params=_SC_PARAMS,
    scratch_types=_COMMON_SCRATCH + [
        pltpu.VMEM_SHARED((NP,), jnp.float32),
        pltpu.SemaphoreType.DMA,
        pltpu.SemaphoreType.DMA,
        pltpu.SemaphoreType.DMA,
    ],
)

_sc_pass2 = pl.kernel(
    functools.partial(_sc_pass_body, False),
    out_type=jax.ShapeDtypeStruct((NC * NP, 64), jnp.float32),
    mesh=_SC_MESH,
    compiler_params=_SC_PARAMS,
    scratch_types=_COMMON_SCRATCH + [
        pltpu.SemaphoreType.DMA,
        pltpu.SemaphoreType.DMA,
        pltpu.SemaphoreType.DMA,
    ],
)


def kernel(features, edge_index, W1, att_src1, att_dst1, W2, gene_att, Wpred, bpred):
    sd = edge_index.astype(jnp.int32).reshape(2 * E)
    features_p = jnp.pad(features, ((0, NP - N), (0, 0)))
    av = jnp.stack([att_src1, att_dst1], axis=1)

    x1p, asdT = _tc1(features_p, gene_att, W1, av)
    a_s = asdT[0]
    a_d = asdT[1]

    h1parts, esumP = _sc_pass1(x1p, sd, a_s, a_d)
    esum2 = esumP.reshape(NC * NROW, 128)
    h2, x3p, lp = _tc2(h1parts, esum2, W2, W2 @ W2.T, W2 @ Wpred,
                       bpred[None, :])
    h3parts = _sc_pass2(x3p, sd, a_s, a_d)
    h4 = _tc3(h3parts, esum2, W1.T)

    return (h2, h4, lp)


# speedup vs baseline: 40.3209x; 1.0460x over previous
import functools

import jax
import jax.numpy as jnp
from jax import lax
from jax.experimental import pallas as pl
from jax.experimental.pallas import tpu as pltpu
from jax.experimental.pallas import tpu_sc as plsc

N = 10000
NP = 10240
E = 320000
NROW = NP // 128
BLKR = 2048
GRIDR = NP // BLKR
BROW = BLKR // 128

NC = 2
NS = 16
NW = NC * NS
CHUNK = 128
RROWS = E // CHUNK
TROWS = 80
NSLC = NP // NS


def _elu(x):
    return jnp.where(x > 0, x, jnp.exp(jnp.minimum(x, 0.0)) - 1.0)


def _tc1_body(f_ref, g_ref, w1_ref, av_ref, x1_ref, asd_ref, c_ref, mx_ref):
    i = pl.program_id(0)
    att = f_ref[...] * g_ref[...]
    x1 = jnp.dot(att, w1_ref[...], preferred_element_type=jnp.float32,
                 precision=lax.Precision.HIGHEST)
    x1_ref[...] = x1
    asdT = lax.dot_general(av_ref[...], x1,
                           dimension_numbers=(((0,), (1,)), ((), ())),
                           preferred_element_type=jnp.float32,
                           precision=lax.Precision.HIGHEST)
    asd_ref[...] = asdT
    m_s = jnp.max(asdT[0, :])
    m_d = jnp.max(asdT[1, :])

    @pl.when(i == 0)
    def _():
        mx_ref[0] = m_s
        mx_ref[1] = m_d

    @pl.when(i > 0)
    def _():
        mx_ref[0] = jnp.maximum(mx_ref[0], m_s)
        mx_ref[1] = jnp.maximum(mx_ref[1], m_d)

    @pl.when(i == GRIDR - 1)
    def _():
        a = mx_ref[0] + mx_ref[1]
        c_ref[...] = jnp.full((1, 128), jnp.where(a >= 0, a, 0.2 * a),
                              jnp.float32)


def _tc1(features_p, gene_att, W1, av):
    return pl.pallas_call(
        _tc1_body,
        grid=(GRIDR,),
        in_specs=[
            pl.BlockSpec((BLKR, 128), lambda i: (i, 0)),
            pl.BlockSpec((1, 128), lambda i: (0, 0)),
            pl.BlockSpec((128, 64), lambda i: (0, 0)),
            pl.BlockSpec((64, 2), lambda i: (0, 0)),
        ],
        out_specs=[
            pl.BlockSpec((BLKR, 64), lambda i: (i, 0)),
            pl.BlockSpec((2, BLKR), lambda i: (0, i)),
            pl.BlockSpec((1, 128), lambda i: (0, 0)),
        ],
        out_shape=[
            jax.ShapeDtypeStruct((NP, 64), jnp.float32),
            jax.ShapeDtypeStruct((2, NP), jnp.float32),
            jax.ShapeDtypeStruct((1, 128), jnp.float32),
        ],
        scratch_shapes=[pltpu.SMEM((2,), jnp.float32)],
    )(features_p, gene_att, W1, av)


def _tc2_body(p0_ref, p1_ref, e0_ref, e1_ref, w2_ref, w2t_ref, wp_ref, bp_ref,
              h2_ref, x3_ref, lp_ref):
    den = e0_ref[...] + e1_ref[...] + 1e-16
    den64 = jnp.reshape(jnp.broadcast_to(den[:, :, None], (BROW, 128, 64)),
                        (BLKR, 64))
    h1 = _elu((p0_ref[...] + p1_ref[...]) / den64)
    h2_ref[...] = jnp.dot(h1, w2_ref[...], preferred_element_type=jnp.float32,
                          precision=lax.Precision.HIGHEST)
    x3_ref[...] = jnp.dot(h1, w2t_ref[...], preferred_element_type=jnp.float32,
                          precision=lax.Precision.HIGHEST)
    pred = jnp.dot(h1, wp_ref[...], preferred_element_type=jnp.float32,
                   precision=lax.Precision.HIGHEST) + bp_ref[...]
    m = jnp.max(pred, axis=-1, keepdims=True)
    lse = jnp.log(jnp.sum(jnp.exp(pred - m), axis=-1, keepdims=True)) + m
    lp_ref[...] = pred - lse


def _tc2(hparts, esum2, W2, W2t, Wpred, bpred2):
    return pl.pallas_call(
        _tc2_body,
        grid=(GRIDR,),
        in_specs=[
            pl.BlockSpec((BLKR, 64), lambda i: (i, 0)),
            pl.BlockSpec((BLKR, 64), lambda i: (i + GRIDR, 0)),
            pl.BlockSpec((BROW, 128), lambda i: (i, 0)),
            pl.BlockSpec((BROW, 128), lambda i: (i + GRIDR, 0)),
            pl.BlockSpec((64, 32), lambda i: (0, 0)),
            pl.BlockSpec((64, 64), lambda i: (0, 0)),
            pl.BlockSpec((64, 16), lambda i: (0, 0)),
            pl.BlockSpec((1, 16), lambda i: (0, 0)),
        ],
        out_specs=[
            pl.BlockSpec((BLKR, 32), lambda i: (i, 0)),
            pl.BlockSpec((BLKR, 64), lambda i: (i, 0)),
            pl.BlockSpec((BLKR, 16), lambda i: (i, 0)),
        ],
        out_shape=[
            jax.ShapeDtypeStruct((N, 32), jnp.float32),
            jax.ShapeDtypeStruct((NP, 64), jnp.float32),
            jax.ShapeDtypeStruct((N, 16), jnp.float32),
        ],
    )(hparts, hparts, esum2, esum2, W2, W2t, Wpred, bpred2)


def _tc3_body(p0_ref, p1_ref, e0_ref, e1_ref, w1t_ref, h4_ref):
    den = e0_ref[...] + e1_ref[...] + 1e-16
    den64 = jnp.reshape(jnp.broadcast_to(den[:, :, None], (BROW, 128, 64)),
                        (BLKR, 64))
    h3 = _elu((p0_ref[...] + p1_ref[...]) / den64)
    h4_ref[...] = jnp.dot(h3, w1t_ref[...], preferred_element_type=jnp.float32,
                          precision=lax.Precision.HIGHEST)


def _tc3(hparts, esum2, W1t):
    return pl.pallas_call(
        _tc3_body,
        grid=(GRIDR,),
        in_specs=[
            pl.BlockSpec((BLKR, 64), lambda i: (i, 0)),
            pl.BlockSpec((BLKR, 64), lambda i: (i + GRIDR, 0)),
            pl.BlockSpec((BROW, 128), lambda i: (i, 0)),
            pl.BlockSpec((BROW, 128), lambda i: (i + GRIDR, 0)),
            pl.BlockSpec((64, 128), lambda i: (0, 0)),
        ],
        out_specs=pl.BlockSpec((BLKR, 128), lambda i: (i, 0)),
        out_shape=jax.ShapeDtypeStruct((N, 128), jnp.float32),
    )(hparts, hparts, esum2, esum2, W1t)


_SC_MESH = plsc.VectorSubcoreMesh(core_axis_name="c", subcore_axis_name="s",
                                  num_cores=NC, num_subcores=NS)
_SC_PARAMS = pltpu.CompilerParams(use_tc_tiling_on_sc=False,
                                  needs_layout_passes=False)


def _wid():
    return lax.axis_index("s") * NC + lax.axis_index("c")


def _gather_tab(tab1, idx16):
    return plsc.load_gather(tab1, [idx16])


def _sc_pass_body(emit_aux, *refs):
    if emit_aux:
        (x_hbm, sd_hbm, as_hbm, ad_hbm, c_hbm, h_hbm, esum_hbm,
         as_v, ad_v, c_v, sd4, ex_v, rows2, h_sh, esum_sh,
         sem_i, sem_g, sem_s) = refs
    else:
        (x_hbm, sd_hbm, as_hbm, ad_hbm, c_hbm, h_hbm,
         as_v, ad_v, c_v, sd4, ex_v, rows2, h_sh,
         sem_i, sem_g, sem_s) = refs
    c = lax.axis_index("c")
    s = lax.axis_index("s")
    wid = _wid()
    tab_s = pltpu.async_copy(as_hbm, as_v, sem_g)
    tab_d = pltpu.async_copy(ad_hbm, ad_v, sem_g)
    pltpu.sync_copy(c_hbm, c_v)
    def zr(i, _):
        for k in range(4):
            rows2[0, i, pl.ds(k * 16, 16)] = jnp.zeros((16,), jnp.float32)
        return 0
    lax.fori_loop(0, CHUNK, zr, 0)
    for r in range(NSLC // CHUNK):
        pltpu.sync_copy(rows2.at[0],
                        h_sh.at[pl.ds(s * NSLC + r * CHUNK, CHUNK)])
    if emit_aux:
        def ze(i, _):
            ex_v[pl.ds(i * 16, 16)] = jnp.zeros((16,), jnp.float32)
            return 0
        lax.fori_loop(0, CHUNK // 16, ze, 0)
        for r in range(NSLC // CHUNK):
            pltpu.sync_copy(ex_v, esum_sh.at[pl.ds(s * NSLC + r * CHUNK, CHUNK)])
    shift = c_v[0, pl.ds(0, 16)]
    tab_s.wait()
    tab_d.wait()
    plsc.subcore_barrier()

    rcnt = jnp.clip(RROWS - wid * TROWS, 0, TROWS)
    base_row = wid * TROWS

    def idx_issue(r, slot):
        pltpu.async_copy(sd_hbm.at[pl.ds(r * CHUNK, CHUNK)],
                         sd4.at[slot, 0], sem_i)
        pltpu.async_copy(sd_hbm.at[pl.ds(E + r * CHUNK, CHUNK)],
                         sd4.at[slot, 1], sem_i)

    def idx_wait(r, slot):
        pltpu.make_async_copy(sd_hbm.at[pl.ds(r * CHUNK, CHUNK)],
                              sd4.at[slot, 0], sem_i).wait()
        pltpu.make_async_copy(sd_hbm.at[pl.ds(E + r * CHUNK, CHUNK)],
                              sd4.at[slot, 1], sem_i).wait()

    def gather_issue(r, slot, p2):
        pltpu.async_copy(x_hbm.at[sd4.at[slot, 0]], rows2.at[p2], sem_g)

    def scat_wait(p2, slot):
        pltpu.make_async_copy(rows2.at[p2], h_sh.at[sd4.at[slot, 1]],
                              sem_s).wait()

    idx_issue(base_row, 0)
    idx_wait(base_row, 0)
    gather_issue(base_row, 0, 0)
    idx_issue(base_row + 1, 1)

    def chunk(j, _):
        p4 = jnp.bitwise_and(j, 3)
        p2 = jnp.bitwise_and(j, 1)
        row = base_row + j
        for k in range(CHUNK // 16):
            si = sd4[p4, 0, pl.ds(k * 16, 16)]
            di = sd4[p4, 1, pl.ds(k * 16, 16)]
            e = _gather_tab(as_v, si) + _gather_tab(ad_v, di)
            e = jnp.where(e >= 0, e, 0.2 * e) - shift
            ex_v[pl.ds(k * 16, 16)] = jnp.exp(e)
        if emit_aux:
            pltpu.sync_copy(ex_v, esum_sh.at[sd4.at[p4, 1]], add=True)

        @pl.when(j + 1 < rcnt)
        def _():
            n4 = jnp.bitwise_and(j + 1, 3)
            n2 = jnp.bitwise_and(j + 1, 1)
            idx_wait(row + 1, n4)

            @pl.when(j >= 1)
            def _():
                scat_wait(n2, jnp.bitwise_and(j - 1, 3))
            gather_issue(row + 1, n4, n2)

            @pl.when(j + 2 < rcnt)
            def _():
                idx_issue(row + 2, jnp.bitwise_and(j + 2, 3))

        pltpu.make_async_copy(x_hbm.at[sd4.at[p4, 0]], rows2.at[p2],
                              sem_g).wait()

        @plsc.parallel_loop(0, CHUNK, step=1, unroll=4)
        def _(e4):
            ab = plsc.load_gather(ex_v, [jnp.full((16,), e4, jnp.int32)])
            for k in range(4):
                rows2[p2, e4, pl.ds(k * 16, 16)] = (
                    rows2[p2, e4, pl.ds(k * 16, 16)] * ab)

        pltpu.async_copy(rows2.at[p2], h_sh.at[sd4.at[p4, 1]], sem_s,
                         add=True)
        return 0

    lax.fori_loop(0, rcnt, chunk, 0)
    for d in range(2):
        pltpu.make_async_copy(rows2.at[d], h_sh.at[sd4.at[d, 1]],
                              sem_s).wait()
    plsc.subcore_barrier()
    pltpu.sync_copy(h_sh.at[pl.ds(s * NSLC, NSLC)],
                    h_hbm.at[pl.ds(c * NP + s * NSLC, NSLC)])
    if emit_aux:
        pltpu.sync_copy(esum_sh.at[pl.ds(s * NSLC, NSLC)],
                        esum_hbm.at[pl.ds(c * NP + s * NSLC, NSLC)])


_COMMON_SCRATCH = [
    pltpu.VMEM((NP,), jnp.float32),
    pltpu.VMEM((NP,), jnp.float32),
    pltpu.VMEM((1, 128), jnp.float32),
    pltpu.VMEM((4, 2, CHUNK), jnp.int32),
    pltpu.VMEM((CHUNK,), jnp.float32),
    pltpu.VMEM((2, CHUNK, 64), jnp.float32),
    pltpu.VMEM_SHARED((NP, 64), jnp.float32),
]

_sc_pass1 = pl.kernel(
    functools.partial(_sc_pass_body, True),
    out_type=[
        jax.ShapeDtypeStruct((NC * NP, 64), jnp.float32),
        jax.ShapeDtypeStruct((NC * NP,), jnp.float32),
    ],
    mesh=_SC_MESH,
    compiler_params=_SC_PARAMS,
    scratch_types=_COMMON_SCRATCH + [
        pltpu.VMEM_SHARED((NP,), jnp.float32),
        pltpu.SemaphoreType.DMA,
        pltpu.SemaphoreType.DMA,
        pltpu.SemaphoreType.DMA,
    ],
)

_sc_pass2 = pl.kernel(
    functools.partial(_sc_pass_body, False),
    out_type=jax.ShapeDtypeStruct((NC * NP, 64), jnp.float32),
    mesh=_SC_MESH,
    compiler_params=_SC_PARAMS,
    scratch_types=_COMMON_SCRATCH + [
        pltpu.SemaphoreType.DMA,
        pltpu.SemaphoreType.DMA,
        pltpu.SemaphoreType.DMA,
    ],
)


def kernel(features, edge_index, W1, att_src1, att_dst1, W2, gene_att, Wpred, bpred):
    sd = edge_index.astype(jnp.int32).reshape(2 * E)
    features_p = jnp.pad(features, ((0, NP - N), (0, 0)))
    av = jnp.stack([att_src1, att_dst1], axis=1)

    x1p, asdT, shiftC = _tc1(features_p, gene_att, W1, av)
    a_s = asdT[0]
    a_d = asdT[1]

    h1parts, esumP = _sc_pass1(x1p, sd, a_s, a_d, shiftC)
    esum2 = esumP.reshape(NC * NROW, 128)
    h2, x3p, lp = _tc2(h1parts, esum2, W2, W2 @ W2.T, W2 @ Wpred,
                       bpred[None, :])
    h3parts = _sc_pass2(x3p, sd, a_s, a_d, shiftC)
    h4 = _tc3(h3parts, esum2, W1.T)

    return (h2, h4, lp)
